# baseline (speedup 1.0000x reference)
import dataclasses
import functools

import jax
import jax.numpy as jnp
from jax import lax
from jax.experimental import pallas as pl
from jax.experimental.pallas import tpu as pltpu
from jax.experimental.pallas import tpu_sc as plsc

N = 10000
NP = 10240
F = 128
H = 64
E = 320000
NS = 16
NW = 32
EPW = 10240
CH = 128
NCH = EPW // CH
EP = NW * EPW
RPS = NP // NS

_MESH = plsc.VectorSubcoreMesh(
    core_axis_name="c", subcore_axis_name="s", num_cores=2, num_subcores=16
)

_SC_PARAMS = pltpu.CompilerParams()
if "needs_layout_passes" in pltpu.CompilerParams.__dataclass_fields__:
    _SC_PARAMS = dataclasses.replace(_SC_PARAMS, needs_layout_passes=False)
if "use_tc_tiling_on_sc" in pltpu.CompilerParams.__dataclass_fields__:
    _SC_PARAMS = dataclasses.replace(_SC_PARAMS, use_tc_tiling_on_sc=False)


def _zero_vec16():
    return jnp.zeros((16,), jnp.float32)


@functools.partial(
    pl.kernel,
    out_type=jax.ShapeDtypeStruct((NW, NP), jnp.float32),
    mesh=_MESH,
    compiler_params=_SC_PARAMS,
    scratch_types=[
        pltpu.VMEM((NCH, CH), jnp.int32),
        pltpu.VMEM((NCH, CH), jnp.float32),
        pltpu.VMEM((NP,), jnp.float32),
    ],
)
def _sc_deg(dst_hbm, w_hbm, out_hbm, dst_v, w_v, acc_v):
    cid = lax.axis_index("c")
    sid = lax.axis_index("s")
    wid = cid * NS + sid

    @pl.loop(0, NP // 16)
    def _(i):
        acc_v[pl.ds(i * 16, 16)] = _zero_vec16()

    pltpu.sync_copy(dst_hbm.at[wid], dst_v)
    pltpu.sync_copy(w_hbm.at[wid], w_v)

    @pl.loop(0, NCH)
    def _(c):
        for j in range(CH // 16):
            idx = dst_v[c, pl.ds(j * 16, 16)]
            val = w_v[c, pl.ds(j * 16, 16)]
            plsc.addupdate_scatter(acc_v, [idx], val)

    pltpu.sync_copy(acc_v, out_hbm.at[wid])


def _make_conv(dim):
    @functools.partial(
        pl.kernel,
        out_type=jax.ShapeDtypeStruct((2, NP, dim), jnp.float32),
        mesh=_MESH,
        compiler_params=_SC_PARAMS,
        scratch_types=[
            pltpu.VMEM((NCH, CH), jnp.int32),
            pltpu.VMEM((NCH, CH), jnp.int32),
            pltpu.VMEM((NCH, CH), jnp.float32),
            pltpu.VMEM((CH, dim), jnp.float32),
            pltpu.VMEM((CH, dim), jnp.float32),
            pltpu.VMEM_SHARED((NP, dim), jnp.float32),
            pltpu.SemaphoreType.DMA,
            pltpu.SemaphoreType.DMA,
        ],
    )
    def conv(h_hbm, src_hbm, dst_hbm, w_hbm, out_hbm,
             src_v, dst_v, w_v, rows0_v, rows1_v, acc_sh, sem0, sem1):
        cid = lax.axis_index("c")
        sid = lax.axis_index("s")
        wid = cid * NS + sid

        @pl.loop(0, CH)
        def _(r):
            for j in range(dim // 16):
                rows0_v[r, pl.ds(j * 16, 16)] = _zero_vec16()

        for k in range(RPS // CH):
            pltpu.sync_copy(rows0_v, acc_sh.at[pl.ds(sid * RPS + k * CH, CH)])
        plsc.subcore_barrier()

        pltpu.sync_copy(src_hbm.at[wid], src_v)
        pltpu.sync_copy(dst_hbm.at[wid], dst_v)
        pltpu.sync_copy(w_hbm.at[wid], w_v)

        def scale(rows_v, c):
            @pl.loop(0, CH // 16)
            def _(g):
                wv = w_v[c, pl.ds(g * 16, 16)]
                for l in range(16):
                    sv = lax.broadcast(wv[l], (16,))
                    for j in range(dim // 16):
                        sl = pl.ds(j * 16, 16)
                        e = g * 16 + l
                        rows_v[e, sl] = rows_v[e, sl] * sv

        if True:
            pltpu.async_copy(h_hbm.at[src_v.at[0]], rows0_v, sem0)
            pltpu.async_copy(h_hbm.at[src_v.at[1]], rows1_v, sem1)

            @pl.loop(0, NCH, step=2)
            def _(c):
                pltpu.make_async_copy(h_hbm.at[src_v.at[c]], rows0_v, sem0).wait()
                scale(rows0_v, c)
                pltpu.sync_copy(rows0_v, acc_sh.at[dst_v.at[c]], add=True)

                @pl.when(c + 2 < NCH)
                def _():
                    pltpu.async_copy(h_hbm.at[src_v.at[c + 2]], rows0_v, sem0)

                pltpu.make_async_copy(h_hbm.at[src_v.at[c + 1]], rows1_v, sem1).wait()
                scale(rows1_v, c + 1)
                pltpu.sync_copy(rows1_v, acc_sh.at[dst_v.at[c + 1]], add=True)

                @pl.when(c + 3 < NCH)
                def _():
                    pltpu.async_copy(h_hbm.at[src_v.at[c + 3]], rows1_v, sem1)
        else:
            @pl.loop(0, NCH)
            def _(c):
                pltpu.async_copy(h_hbm.at[src_v.at[c]], rows0_v, sem0).wait()
                scale(rows0_v, c)
                pltpu.sync_copy(rows0_v, acc_sh.at[dst_v.at[c]], add=True)

        plsc.subcore_barrier()
        for k in range(RPS // CH):
            sl = pl.ds(sid * RPS + k * CH, CH)
            pltpu.sync_copy(acc_sh.at[sl], out_hbm.at[cid].at[sl])

    return conv


_sc_conv64 = _make_conv(64)


@functools.partial(
    pl.kernel,
    out_type=(
        jax.ShapeDtypeStruct((NW, NCH, CH), jnp.float32),
        jax.ShapeDtypeStruct((NW, NP), jnp.float32),
    ),
    mesh=_MESH,
    compiler_params=_SC_PARAMS,
    scratch_types=[
        pltpu.VMEM((NCH, CH), jnp.int32),
        pltpu.VMEM((NCH, CH), jnp.int32),
        pltpu.VMEM((NCH, CH), jnp.float32),
        pltpu.VMEM((CH, F), jnp.float32),
        pltpu.VMEM((CH, F), jnp.float32),
        pltpu.VMEM((CH, F), jnp.float32),
        pltpu.VMEM((CH, F), jnp.float32),
        pltpu.VMEM((NP,), jnp.float32),
        pltpu.SemaphoreType.DMA,
        pltpu.SemaphoreType.DMA,
        pltpu.SemaphoreType.DMA,
        pltpu.SemaphoreType.DMA,
    ],
)
def _sc_decode(z_hbm, src_hbm, dst_hbm, probs_hbm, degp_hbm,
               src_v, dst_v, probs_v, zs0_v, zd0_v, zs1_v, zd1_v,
               acc_v, ss0, sd0, ss1, sd1):
    cid = lax.axis_index("c")
    sid = lax.axis_index("s")
    wid = cid * NS + sid

    @pl.loop(0, NP // 16)
    def _(i):
        acc_v[pl.ds(i * 16, 16)] = _zero_vec16()

    pltpu.sync_copy(src_hbm.at[wid], src_v)
    pltpu.sync_copy(dst_hbm.at[wid], dst_v)

    lane = lax.iota(jnp.int32, 16)
    ebase = wid * EPW

    def dots_chunk(zs_v, zd_v, c):
        @pl.loop(0, CH // 16)
        def _(g):
            dots = _zero_vec16()
            for l in range(16):
                e = g * 16 + l
                prod = zs_v[e, pl.ds(0, 16)] * zd_v[e, pl.ds(0, 16)]
                for j in range(1, F // 16):
                    sl = pl.ds(j * 16, 16)
                    prod = prod + zs_v[e, sl] * zd_v[e, sl]
                dots = dots + jnp.where(lane == l, jnp.sum(prod), 0.0)
            sl = pl.ds(g * 16, 16)
            eid = lane + (ebase + c * CH + g * 16)
            p = jnp.where(eid < E, 1.0 / (1.0 + jnp.exp(-dots)), 0.0)
            probs_v[c, sl] = p
            plsc.addupdate_scatter(acc_v, [dst_v[c, sl]], p)

    pltpu.async_copy(z_hbm.at[src_v.at[0]], zs0_v, ss0)
    pltpu.async_copy(z_hbm.at[dst_v.at[0]], zd0_v, sd0)
    pltpu.async_copy(z_hbm.at[src_v.at[1]], zs1_v, ss1)
    pltpu.async_copy(z_hbm.at[dst_v.at[1]], zd1_v, sd1)

    @pl.loop(0, NCH, step=2)
    def _(c):
        pltpu.make_async_copy(z_hbm.at[src_v.at[c]], zs0_v, ss0).wait()
        pltpu.make_async_copy(z_hbm.at[dst_v.at[c]], zd0_v, sd0).wait()
        dots_chunk(zs0_v, zd0_v, c)

        @pl.when(c + 2 < NCH)
        def _():
            pltpu.async_copy(z_hbm.at[src_v.at[c + 2]], zs0_v, ss0)
            pltpu.async_copy(z_hbm.at[dst_v.at[c + 2]], zd0_v, sd0)

        pltpu.make_async_copy(z_hbm.at[src_v.at[c + 1]], zs1_v, ss1).wait()
        pltpu.make_async_copy(z_hbm.at[dst_v.at[c + 1]], zd1_v, sd1).wait()
        dots_chunk(zs1_v, zd1_v, c + 1)

        @pl.when(c + 3 < NCH)
        def _():
            pltpu.async_copy(z_hbm.at[src_v.at[c + 3]], zs1_v, ss1)
            pltpu.async_copy(z_hbm.at[dst_v.at[c + 3]], zd1_v, sd1)

    pltpu.sync_copy(probs_v, probs_hbm.at[wid])
    pltpu.sync_copy(acc_v, degp_hbm.at[wid])


_BT = 512


def _mm(x, W):
    M, K = x.shape
    _, Nn = W.shape

    def body(x_ref, w_ref, o_ref):
        o_ref[...] = jnp.dot(x_ref[...], w_ref[...],
                             preferred_element_type=jnp.float32)

    return pl.pallas_call(
        body,
        grid=(M // _BT,),
        in_specs=[pl.BlockSpec((_BT, K), lambda i: (i, 0)),
                  pl.BlockSpec((K, Nn), lambda i: (0, 0))],
        out_specs=pl.BlockSpec((_BT, Nn), lambda i: (i, 0)),
        out_shape=jax.ShapeDtypeStruct((M, Nn), jnp.float32),
    )(x, W)


def _norms(degp, h1):
    def body(dp_ref, h1_ref, h1p_ref, dis_ref, invd_ref):
        deg = jnp.sum(dp_ref[...], axis=0, keepdims=True) + 1.0
        dis = lax.rsqrt(deg)
        invd = 1.0 / deg
        dis_ref[...] = dis.T
        invd_ref[...] = invd.T
        h1p_ref[...] = h1_ref[...] * dis.T

    return pl.pallas_call(
        body,
        grid=(NP // _BT,),
        in_specs=[pl.BlockSpec((NW, _BT), lambda i: (0, i)),
                  pl.BlockSpec((_BT, H), lambda i: (i, 0))],
        out_specs=[pl.BlockSpec((_BT, H), lambda i: (i, 0)),
                   pl.BlockSpec((_BT, 1), lambda i: (i, 0)),
                   pl.BlockSpec((_BT, 1), lambda i: (i, 0))],
        out_shape=[jax.ShapeDtypeStruct((NP, H), jnp.float32),
                   jax.ShapeDtypeStruct((NP, 1), jnp.float32),
                   jax.ShapeDtypeStruct((NP, 1), jnp.float32)],
    )(degp, h1)


def _post1(acc, h1, dis, invd, b1, W2):
    def body(acc_ref, h1_ref, dis_ref, invd_ref, b1_ref, w2_ref,
             h2pa_ref, h2pb_ref, h2i_ref):
        s = acc_ref[0] + acc_ref[1]
        h = jnp.maximum(
            dis_ref[...] * s + h1_ref[...] * invd_ref[...] + b1_ref[...], 0.0)
        h2 = jnp.dot(h, w2_ref[...], preferred_element_type=jnp.float32)
        h2p = h2 * dis_ref[...]
        h2pa_ref[...] = h2p[:, :H]
        h2pb_ref[...] = h2p[:, H:]
        h2i_ref[...] = h2 * invd_ref[...]

    return pl.pallas_call(
        body,
        grid=(NP // _BT,),
        in_specs=[pl.BlockSpec((2, _BT, H), lambda i: (0, i, 0)),
                  pl.BlockSpec((_BT, H), lambda i: (i, 0)),
                  pl.BlockSpec((_BT, 1), lambda i: (i, 0)),
                  pl.BlockSpec((_BT, 1), lambda i: (i, 0)),
                  pl.BlockSpec((1, H), lambda i: (0, 0)),
                  pl.BlockSpec((H, F), lambda i: (0, 0))],
        out_specs=[pl.BlockSpec((_BT, H), lambda i: (i, 0)),
                   pl.BlockSpec((_BT, H), lambda i: (i, 0)),
                   pl.BlockSpec((_BT, F), lambda i: (i, 0))],
        out_shape=[jax.ShapeDtypeStruct((NP, H), jnp.float32),
                   jax.ShapeDtypeStruct((NP, H), jnp.float32),
                   jax.ShapeDtypeStruct((NP, F), jnp.float32)],
    )(acc, h1, dis, invd, b1, W2)


def _post2(acca, accb, h2i, dis, b2, x, Wd):
    def body(acca_ref, accb_ref, h2i_ref, dis_ref, b2_ref, x_ref, wd_ref,
             z_ref, y_ref):
        s = jnp.concatenate([acca_ref[0] + acca_ref[1],
                             accb_ref[0] + accb_ref[1]], axis=1)
        logits = dis_ref[...] * s + h2i_ref[...] + b2_ref[...]
        z = 1.0 / (1.0 + jnp.exp(-logits))
        z_ref[...] = z
        y_ref[...] = jnp.dot(z + x_ref[...], wd_ref[...],
                             preferred_element_type=jnp.float32)

    return pl.pallas_call(
        body,
        grid=(NP // _BT,),
        in_specs=[pl.BlockSpec((2, _BT, H), lambda i: (0, i, 0)),
                  pl.BlockSpec((2, _BT, H), lambda i: (0, i, 0)),
                  pl.BlockSpec((_BT, F), lambda i: (i, 0)),
                  pl.BlockSpec((_BT, 1), lambda i: (i, 0)),
                  pl.BlockSpec((1, F), lambda i: (0, 0)),
                  pl.BlockSpec((_BT, F), lambda i: (i, 0)),
                  pl.BlockSpec((F, H), lambda i: (0, 0))],
        out_specs=[pl.BlockSpec((_BT, F), lambda i: (i, 0)),
                   pl.BlockSpec((_BT, H), lambda i: (i, 0))],
        out_shape=[jax.ShapeDtypeStruct((NP, F), jnp.float32),
                   jax.ShapeDtypeStruct((NP, H), jnp.float32)],
    )(acca, accb, h2i, dis, b2, x, Wd)


def _norms2(degp, y):
    def body(dp_ref, y_ref, yp_ref, yi_ref, dis_ref):
        deg = jnp.sum(dp_ref[...], axis=0, keepdims=True) + 1.0
        dis = lax.rsqrt(deg)
        invd = 1.0 / deg
        yp_ref[...] = y_ref[...] * dis.T
        yi_ref[...] = y_ref[...] * invd.T
        dis_ref[...] = dis.T

    return pl.pallas_call(
        body,
        grid=(NP // _BT,),
        in_specs=[pl.BlockSpec((NW, _BT), lambda i: (0, i)),
                  pl.BlockSpec((_BT, H), lambda i: (i, 0))],
        out_specs=[pl.BlockSpec((_BT, H), lambda i: (i, 0)),
                   pl.BlockSpec((_BT, H), lambda i: (i, 0)),
                   pl.BlockSpec((_BT, 1), lambda i: (i, 0))],
        out_shape=[jax.ShapeDtypeStruct((NP, H), jnp.float32),
                   jax.ShapeDtypeStruct((NP, H), jnp.float32),
                   jax.ShapeDtypeStruct((NP, 1), jnp.float32)],
    )(degp, y)


def _head(acc, yi, dis2, bd, Wfc2d, bfc):
    nsteps = NP // _BT

    def body(acc_ref, yi_ref, dis_ref, bd_ref, wfc_ref, bfc_ref, o_ref):
        i = pl.program_id(0)

        @pl.when(i == 0)
        def _():
            o_ref[...] = jnp.zeros((1, 1), jnp.float32)

        s = acc_ref[0] + acc_ref[1]
        hd = jnp.maximum(
            dis_ref[...] * s + yi_ref[...] + bd_ref[...], 0.0)
        o_ref[...] += jnp.sum(hd * wfc_ref[...]).reshape(1, 1)

        @pl.when(i == nsteps - 1)
        def _():
            t = o_ref[...] + bfc_ref[...]
            o_ref[...] = 1.0 / (1.0 + jnp.exp(-t))

    return pl.pallas_call(
        body,
        grid=(nsteps,),
        in_specs=[pl.BlockSpec((2, _BT, H), lambda i: (0, i, 0)),
                  pl.BlockSpec((_BT, H), lambda i: (i, 0)),
                  pl.BlockSpec((_BT, 1), lambda i: (i, 0)),
                  pl.BlockSpec((1, H), lambda i: (0, 0)),
                  pl.BlockSpec((_BT, H), lambda i: (i, 0)),
                  pl.BlockSpec((1, 1), lambda i: (0, 0))],
        out_specs=pl.BlockSpec((1, 1), lambda i: (0, 0)),
        out_shape=jax.ShapeDtypeStruct((1, 1), jnp.float32),
    )(acc, yi, dis2, bd, Wfc2d, bfc)


def kernel(features, edge_index, edge_attr, W1, b1, W2, b2, Wd, bd, Wfc, bfc):
    pad = EP - E
    src = jnp.concatenate(
        [edge_index[0].astype(jnp.int32), jnp.zeros((pad,), jnp.int32)]
    ).reshape(NW, NCH, CH)
    dst = jnp.concatenate(
        [edge_index[1].astype(jnp.int32), jnp.zeros((pad,), jnp.int32)]
    ).reshape(NW, NCH, CH)
    w = jnp.concatenate(
        [edge_attr.astype(jnp.float32), jnp.zeros((pad,), jnp.float32)]
    ).reshape(NW, NCH, CH)
    xP = jnp.pad(features, ((0, NP - N), (0, 0)))
    WfcP = jnp.pad(Wfc.reshape(N, H), ((0, NP - N), (0, 0)))
    b1r = b1.reshape(1, H)
    b2r = b2.reshape(1, F)
    bdr = bd.reshape(1, H)
    bfcr = bfc.reshape(1, 1)

    degp = _sc_deg(dst, w)
    h1 = _mm(xP, W1)
    h1p, dis, invd = _norms(degp, h1)
    acc1 = _sc_conv64(h1p, src, dst, w)
    h2pa, h2pb, h2i = _post1(acc1, h1, dis, invd, b1r, W2)
    acc2a = _sc_conv64(h2pa, src, dst, w)
    acc2b = _sc_conv64(h2pb, src, dst, w)
    z, y = _post2(acc2a, acc2b, h2i, dis, b2r, xP, Wd)
    probs, degp2 = _sc_decode(z, src, dst)
    yp, yi, dis2 = _norms2(degp2, y)
    acc3 = _sc_conv64(yp, src, dst, probs)
    out = _head(acc3, yi, dis2, bdr, WfcP, bfcr)
    return out[0, 0]


# speedup vs baseline: 1.0935x; 1.0935x over previous
import dataclasses
import functools

import jax
import jax.numpy as jnp
from jax import lax
from jax.experimental import pallas as pl
from jax.experimental.pallas import tpu as pltpu
from jax.experimental.pallas import tpu_sc as plsc

N = 10000
NP = 10240
F = 128
H = 64
E = 320000
NS = 16
NW = 32
EPW = 10240
CH = 128
NCH = EPW // CH
EP = NW * EPW
RPS = NP // NS

_MESH = plsc.VectorSubcoreMesh(
    core_axis_name="c", subcore_axis_name="s", num_cores=2, num_subcores=16
)

_SC_PARAMS = pltpu.CompilerParams()
if "needs_layout_passes" in pltpu.CompilerParams.__dataclass_fields__:
    _SC_PARAMS = dataclasses.replace(_SC_PARAMS, needs_layout_passes=False)
if "use_tc_tiling_on_sc" in pltpu.CompilerParams.__dataclass_fields__:
    _SC_PARAMS = dataclasses.replace(_SC_PARAMS, use_tc_tiling_on_sc=False)


def _zero_vec16():
    return jnp.zeros((16,), jnp.float32)


@functools.partial(
    pl.kernel,
    out_type=jax.ShapeDtypeStruct((NW, NP), jnp.float32),
    mesh=_MESH,
    compiler_params=_SC_PARAMS,
    scratch_types=[
        pltpu.VMEM((NCH, CH), jnp.int32),
        pltpu.VMEM((NCH, CH), jnp.float32),
        pltpu.VMEM((NP,), jnp.float32),
    ],
)
def _sc_deg(dst_hbm, w_hbm, out_hbm, dst_v, w_v, acc_v):
    cid = lax.axis_index("c")
    sid = lax.axis_index("s")
    wid = cid * NS + sid

    @pl.loop(0, NP // 16)
    def _(i):
        acc_v[pl.ds(i * 16, 16)] = _zero_vec16()

    pltpu.sync_copy(dst_hbm.at[wid], dst_v)
    pltpu.sync_copy(w_hbm.at[wid], w_v)

    @pl.loop(0, NCH)
    def _(c):
        for j in range(CH // 16):
            idx = dst_v[c, pl.ds(j * 16, 16)]
            val = w_v[c, pl.ds(j * 16, 16)]
            plsc.addupdate_scatter(acc_v, [idx], val)

    pltpu.sync_copy(acc_v, out_hbm.at[wid])


def _make_conv(dim):
    nbuf = 4 if dim <= 64 else 1
    scratch = [
        pltpu.VMEM((NCH, CH), jnp.int32),
        pltpu.VMEM((NCH, CH), jnp.int32),
        pltpu.VMEM((NCH, CH), jnp.float32),
    ]
    scratch += [pltpu.VMEM((CH, dim), jnp.float32) for _ in range(nbuf)]
    scratch += [pltpu.VMEM_SHARED((NP, dim), jnp.float32)]
    scratch += [pltpu.SemaphoreType.DMA for _ in range(2 * nbuf)]

    @functools.partial(
        pl.kernel,
        out_type=jax.ShapeDtypeStruct((2, NP, dim), jnp.float32),
        mesh=_MESH,
        compiler_params=_SC_PARAMS,
        scratch_types=scratch,
    )
    def conv(h_hbm, src_hbm, dst_hbm, w_hbm, out_hbm, src_v, dst_v, w_v,
             *rest):
        bufs = rest[:nbuf]
        acc_sh = rest[nbuf]
        gsems = rest[nbuf + 1:2 * nbuf + 1]
        ssems = rest[2 * nbuf + 1:]
        cid = lax.axis_index("c")
        sid = lax.axis_index("s")
        wid = cid * NS + sid

        @pl.loop(0, CH)
        def _(r):
            for j in range(dim // 16):
                bufs[0][r, pl.ds(j * 16, 16)] = _zero_vec16()

        for k in range(RPS // CH):
            pltpu.sync_copy(bufs[0], acc_sh.at[pl.ds(sid * RPS + k * CH, CH)])
        plsc.subcore_barrier()

        pltpu.sync_copy(src_hbm.at[wid], src_v)
        pltpu.sync_copy(dst_hbm.at[wid], dst_v)
        pltpu.sync_copy(w_hbm.at[wid], w_v)

        def scale(rows_v, c):
            @pl.loop(0, CH // 16)
            def _(g):
                wv = w_v[c, pl.ds(g * 16, 16)]
                for l in range(16):
                    sv = lax.broadcast(wv[l], (16,))
                    for j in range(dim // 16):
                        sl = pl.ds(j * 16, 16)
                        e = g * 16 + l
                        rows_v[e, sl] = rows_v[e, sl] * sv

        if nbuf == 4:
            for b in range(4):
                pltpu.async_copy(h_hbm.at[src_v.at[b]], bufs[b], gsems[b])

            @pl.loop(0, NCH, step=4)
            def _(c):
                for k in range(4):
                    cc = c + k
                    kp = (k - 1) % 4
                    pltpu.make_async_copy(
                        h_hbm.at[src_v.at[cc]], bufs[k], gsems[k]).wait()
                    scale(bufs[k], cc)
                    pltpu.async_copy(
                        bufs[k], acc_sh.at[dst_v.at[cc]], ssems[k], add=True)

                    @pl.when((cc >= 1) & (cc + 3 < NCH))
                    def _(kp=kp, cc=cc):
                        pltpu.make_async_copy(
                            bufs[kp], acc_sh.at[dst_v.at[cc - 1]],
                            ssems[kp]).wait()
                        pltpu.async_copy(
                            h_hbm.at[src_v.at[cc + 3]], bufs[kp], gsems[kp])

            for k in range(4):
                pltpu.make_async_copy(
                    bufs[k], acc_sh.at[dst_v.at[NCH - 4 + k]],
                    ssems[k]).wait()
        else:
            @pl.loop(0, NCH)
            def _(c):
                pltpu.async_copy(h_hbm.at[src_v.at[c]], bufs[0], gsems[0]).wait()
                scale(bufs[0], c)
                pltpu.sync_copy(bufs[0], acc_sh.at[dst_v.at[c]], add=True)

        plsc.subcore_barrier()
        for k in range(RPS // CH):
            sl = pl.ds(sid * RPS + k * CH, CH)
            pltpu.sync_copy(acc_sh.at[sl], out_hbm.at[cid].at[sl])

    return conv


_sc_conv64 = _make_conv(64)
_sc_conv128 = _make_conv(128)


@functools.partial(
    pl.kernel,
    out_type=(
        jax.ShapeDtypeStruct((NW, NCH, CH), jnp.float32),
        jax.ShapeDtypeStruct((NW, NP), jnp.float32),
    ),
    mesh=_MESH,
    compiler_params=_SC_PARAMS,
    scratch_types=[
        pltpu.VMEM((NCH, CH), jnp.int32),
        pltpu.VMEM((NCH, CH), jnp.int32),
        pltpu.VMEM((NCH, CH), jnp.float32),
        pltpu.VMEM((CH, F), jnp.float32),
        pltpu.VMEM((CH, F), jnp.float32),
        pltpu.VMEM((CH, F), jnp.float32),
        pltpu.VMEM((CH, F), jnp.float32),
        pltpu.VMEM((NP,), jnp.float32),
        pltpu.SemaphoreType.DMA,
        pltpu.SemaphoreType.DMA,
        pltpu.SemaphoreType.DMA,
        pltpu.SemaphoreType.DMA,
    ],
)
def _sc_decode(z_hbm, src_hbm, dst_hbm, probs_hbm, degp_hbm,
               src_v, dst_v, probs_v, zs0_v, zd0_v, zs1_v, zd1_v,
               acc_v, ss0, sd0, ss1, sd1):
    cid = lax.axis_index("c")
    sid = lax.axis_index("s")
    wid = cid * NS + sid

    @pl.loop(0, NP // 16)
    def _(i):
        acc_v[pl.ds(i * 16, 16)] = _zero_vec16()

    pltpu.sync_copy(src_hbm.at[wid], src_v)
    pltpu.sync_copy(dst_hbm.at[wid], dst_v)

    lane = lax.iota(jnp.int32, 16)
    ebase = wid * EPW

    perms = {m: lane ^ m for m in (1, 2, 4, 8)}
    masks = {m: (lane & m) == 0 for m in (1, 2, 4, 8)}
    _dn = lax.GatherDimensionNumbers(
        offset_dims=(), collapsed_slice_dims=(0,), start_index_map=(0,))

    def _perm(v, idx):
        return lax.gather(v, idx[:, None], _dn, (1,),
                          mode=lax.GatherScatterMode.PROMISE_IN_BOUNDS)

    def dots_chunk(zs_v, zd_v, c):
        @pl.loop(0, CH // 16)
        def _(g):
            ps = []
            for l in range(16):
                e = g * 16 + l
                prod = zs_v[e, pl.ds(0, 16)] * zd_v[e, pl.ds(0, 16)]
                for j in range(1, F // 16):
                    sl = pl.ds(j * 16, 16)
                    prod = prod + zs_v[e, sl] * zd_v[e, sl]
                ps.append(prod)
            for m in (1, 2, 4, 8):
                nxt = []
                for i in range(0, len(ps), 2):
                    a2 = ps[i] + _perm(ps[i], perms[m])
                    b2 = ps[i + 1] + _perm(ps[i + 1], perms[m])
                    nxt.append(jnp.where(masks[m], a2, b2))
                ps = nxt
            dots = ps[0]
            sl = pl.ds(g * 16, 16)
            eid = lane + (ebase + c * CH + g * 16)
            p = jnp.where(eid < E, 1.0 / (1.0 + jnp.exp(-dots)), 0.0)
            probs_v[c, sl] = p
            plsc.addupdate_scatter(acc_v, [dst_v[c, sl]], p)

    pltpu.async_copy(z_hbm.at[src_v.at[0]], zs0_v, ss0)
    pltpu.async_copy(z_hbm.at[dst_v.at[0]], zd0_v, sd0)
    pltpu.async_copy(z_hbm.at[src_v.at[1]], zs1_v, ss1)
    pltpu.async_copy(z_hbm.at[dst_v.at[1]], zd1_v, sd1)

    @pl.loop(0, NCH, step=2)
    def _(c):
        pltpu.make_async_copy(z_hbm.at[src_v.at[c]], zs0_v, ss0).wait()
        pltpu.make_async_copy(z_hbm.at[dst_v.at[c]], zd0_v, sd0).wait()
        dots_chunk(zs0_v, zd0_v, c)

        @pl.when(c + 2 < NCH)
        def _():
            pltpu.async_copy(z_hbm.at[src_v.at[c + 2]], zs0_v, ss0)
            pltpu.async_copy(z_hbm.at[dst_v.at[c + 2]], zd0_v, sd0)

        pltpu.make_async_copy(z_hbm.at[src_v.at[c + 1]], zs1_v, ss1).wait()
        pltpu.make_async_copy(z_hbm.at[dst_v.at[c + 1]], zd1_v, sd1).wait()
        dots_chunk(zs1_v, zd1_v, c + 1)

        @pl.when(c + 3 < NCH)
        def _():
            pltpu.async_copy(z_hbm.at[src_v.at[c + 3]], zs1_v, ss1)
            pltpu.async_copy(z_hbm.at[dst_v.at[c + 3]], zd1_v, sd1)

    pltpu.sync_copy(probs_v, probs_hbm.at[wid])
    pltpu.sync_copy(acc_v, degp_hbm.at[wid])


_BT = 512


def _mm(x, W):
    M, K = x.shape
    _, Nn = W.shape

    def body(x_ref, w_ref, o_ref):
        o_ref[...] = jnp.dot(x_ref[...], w_ref[...],
                             preferred_element_type=jnp.float32)

    return pl.pallas_call(
        body,
        grid=(M // _BT,),
        in_specs=[pl.BlockSpec((_BT, K), lambda i: (i, 0)),
                  pl.BlockSpec((K, Nn), lambda i: (0, 0))],
        out_specs=pl.BlockSpec((_BT, Nn), lambda i: (i, 0)),
        out_shape=jax.ShapeDtypeStruct((M, Nn), jnp.float32),
    )(x, W)


def _norms(degp, h1):
    def body(dp_ref, h1_ref, h1p_ref, dis_ref, invd_ref):
        deg = jnp.sum(dp_ref[...], axis=0, keepdims=True) + 1.0
        dis = lax.rsqrt(deg)
        invd = 1.0 / deg
        dis_ref[...] = dis.T
        invd_ref[...] = invd.T
        h1p_ref[...] = h1_ref[...] * dis.T

    return pl.pallas_call(
        body,
        grid=(NP // _BT,),
        in_specs=[pl.BlockSpec((NW, _BT), lambda i: (0, i)),
                  pl.BlockSpec((_BT, H), lambda i: (i, 0))],
        out_specs=[pl.BlockSpec((_BT, H), lambda i: (i, 0)),
                   pl.BlockSpec((_BT, 1), lambda i: (i, 0)),
                   pl.BlockSpec((_BT, 1), lambda i: (i, 0))],
        out_shape=[jax.ShapeDtypeStruct((NP, H), jnp.float32),
                   jax.ShapeDtypeStruct((NP, 1), jnp.float32),
                   jax.ShapeDtypeStruct((NP, 1), jnp.float32)],
    )(degp, h1)


def _post1(acc, h1, dis, invd, b1, W2):
    def body(acc_ref, h1_ref, dis_ref, invd_ref, b1_ref, w2_ref,
             h2p_ref, h2i_ref):
        s = acc_ref[0] + acc_ref[1]
        h = jnp.maximum(
            dis_ref[...] * s + h1_ref[...] * invd_ref[...] + b1_ref[...], 0.0)
        h2 = jnp.dot(h, w2_ref[...], preferred_element_type=jnp.float32)
        h2p_ref[...] = h2 * dis_ref[...]
        h2i_ref[...] = h2 * invd_ref[...]

    return pl.pallas_call(
        body,
        grid=(NP // _BT,),
        in_specs=[pl.BlockSpec((2, _BT, H), lambda i: (0, i, 0)),
                  pl.BlockSpec((_BT, H), lambda i: (i, 0)),
                  pl.BlockSpec((_BT, 1), lambda i: (i, 0)),
                  pl.BlockSpec((_BT, 1), lambda i: (i, 0)),
                  pl.BlockSpec((1, H), lambda i: (0, 0)),
                  pl.BlockSpec((H, F), lambda i: (0, 0))],
        out_specs=[pl.BlockSpec((_BT, F), lambda i: (i, 0)),
                   pl.BlockSpec((_BT, F), lambda i: (i, 0))],
        out_shape=[jax.ShapeDtypeStruct((NP, F), jnp.float32),
                   jax.ShapeDtypeStruct((NP, F), jnp.float32)],
    )(acc, h1, dis, invd, b1, W2)


def _post2(acc, h2i, dis, b2, x, Wd):
    def body(acc_ref, h2i_ref, dis_ref, b2_ref, x_ref, wd_ref, z_ref, y_ref):
        s = acc_ref[0] + acc_ref[1]
        logits = dis_ref[...] * s + h2i_ref[...] + b2_ref[...]
        z = 1.0 / (1.0 + jnp.exp(-logits))
        z_ref[...] = z
        y_ref[...] = jnp.dot(z + x_ref[...], wd_ref[...],
                             preferred_element_type=jnp.float32)

    return pl.pallas_call(
        body,
        grid=(NP // _BT,),
        in_specs=[pl.BlockSpec((2, _BT, F), lambda i: (0, i, 0)),
                  pl.BlockSpec((_BT, F), lambda i: (i, 0)),
                  pl.BlockSpec((_BT, 1), lambda i: (i, 0)),
                  pl.BlockSpec((1, F), lambda i: (0, 0)),
                  pl.BlockSpec((_BT, F), lambda i: (i, 0)),
                  pl.BlockSpec((F, H), lambda i: (0, 0))],
        out_specs=[pl.BlockSpec((_BT, F), lambda i: (i, 0)),
                   pl.BlockSpec((_BT, H), lambda i: (i, 0))],
        out_shape=[jax.ShapeDtypeStruct((NP, F), jnp.float32),
                   jax.ShapeDtypeStruct((NP, H), jnp.float32)],
    )(acc, h2i, dis, b2, x, Wd)


def _norms2(degp, y):
    def body(dp_ref, y_ref, yp_ref, yi_ref, dis_ref):
        deg = jnp.sum(dp_ref[...], axis=0, keepdims=True) + 1.0
        dis = lax.rsqrt(deg)
        invd = 1.0 / deg
        yp_ref[...] = y_ref[...] * dis.T
        yi_ref[...] = y_ref[...] * invd.T
        dis_ref[...] = dis.T

    return pl.pallas_call(
        body,
        grid=(NP // _BT,),
        in_specs=[pl.BlockSpec((NW, _BT), lambda i: (0, i)),
                  pl.BlockSpec((_BT, H), lambda i: (i, 0))],
        out_specs=[pl.BlockSpec((_BT, H), lambda i: (i, 0)),
                   pl.BlockSpec((_BT, H), lambda i: (i, 0)),
                   pl.BlockSpec((_BT, 1), lambda i: (i, 0))],
        out_shape=[jax.ShapeDtypeStruct((NP, H), jnp.float32),
                   jax.ShapeDtypeStruct((NP, H), jnp.float32),
                   jax.ShapeDtypeStruct((NP, 1), jnp.float32)],
    )(degp, y)


def _head(acc, yi, dis2, bd, Wfc2d, bfc):
    nsteps = NP // _BT

    def body(acc_ref, yi_ref, dis_ref, bd_ref, wfc_ref, bfc_ref, o_ref):
        i = pl.program_id(0)

        @pl.when(i == 0)
        def _():
            o_ref[...] = jnp.zeros((1, 1), jnp.float32)

        s = acc_ref[0] + acc_ref[1]
        hd = jnp.maximum(
            dis_ref[...] * s + yi_ref[...] + bd_ref[...], 0.0)
        o_ref[...] += jnp.sum(hd * wfc_ref[...]).reshape(1, 1)

        @pl.when(i == nsteps - 1)
        def _():
            t = o_ref[...] + bfc_ref[...]
            o_ref[...] = 1.0 / (1.0 + jnp.exp(-t))

    return pl.pallas_call(
        body,
        grid=(nsteps,),
        in_specs=[pl.BlockSpec((2, _BT, H), lambda i: (0, i, 0)),
                  pl.BlockSpec((_BT, H), lambda i: (i, 0)),
                  pl.BlockSpec((_BT, 1), lambda i: (i, 0)),
                  pl.BlockSpec((1, H), lambda i: (0, 0)),
                  pl.BlockSpec((_BT, H), lambda i: (i, 0)),
                  pl.BlockSpec((1, 1), lambda i: (0, 0))],
        out_specs=pl.BlockSpec((1, 1), lambda i: (0, 0)),
        out_shape=jax.ShapeDtypeStruct((1, 1), jnp.float32),
    )(acc, yi, dis2, bd, Wfc2d, bfc)


def kernel(features, edge_index, edge_attr, W1, b1, W2, b2, Wd, bd, Wfc, bfc):
    pad = EP - E
    src = jnp.concatenate(
        [edge_index[0].astype(jnp.int32), jnp.zeros((pad,), jnp.int32)]
    ).reshape(NW, NCH, CH)
    dst = jnp.concatenate(
        [edge_index[1].astype(jnp.int32), jnp.zeros((pad,), jnp.int32)]
    ).reshape(NW, NCH, CH)
    w = jnp.concatenate(
        [edge_attr.astype(jnp.float32), jnp.zeros((pad,), jnp.float32)]
    ).reshape(NW, NCH, CH)
    xP = jnp.pad(features, ((0, NP - N), (0, 0)))
    WfcP = jnp.pad(Wfc.reshape(N, H), ((0, NP - N), (0, 0)))
    b1r = b1.reshape(1, H)
    b2r = b2.reshape(1, F)
    bdr = bd.reshape(1, H)
    bfcr = bfc.reshape(1, 1)

    degp = _sc_deg(dst, w)
    h1 = _mm(xP, W1)
    h1p, dis, invd = _norms(degp, h1)
    acc1 = _sc_conv64(h1p, src, dst, w)
    h2p, h2i = _post1(acc1, h1, dis, invd, b1r, W2)
    acc2 = _sc_conv128(h2p, src, dst, w)
    z, y = _post2(acc2, h2i, dis, b2r, xP, Wd)
    probs, degp2 = _sc_decode(z, src, dst)
    yp, yi, dis2 = _norms2(degp2, y)
    acc3 = _sc_conv64(yp, src, dst, probs)
    out = _head(acc3, yi, dis2, bdr, WfcP, bfcr)
    return out[0, 0]


# speedup vs baseline: 1.3420x; 1.2273x over previous
import dataclasses
import functools

import jax
import jax.numpy as jnp
from jax import lax
from jax.experimental import pallas as pl
from jax.experimental.pallas import tpu as pltpu
from jax.experimental.pallas import tpu_sc as plsc

N = 10000
NP = 10240
F = 128
H = 64
E = 320000
NS = 16
NW = 32
EPW = 10240
CH = 128
NCH = EPW // CH
EP = NW * EPW
RPS = NP // NS

_MESH = plsc.VectorSubcoreMesh(
    core_axis_name="c", subcore_axis_name="s", num_cores=2, num_subcores=16
)

_SC_PARAMS = pltpu.CompilerParams()
if "needs_layout_passes" in pltpu.CompilerParams.__dataclass_fields__:
    _SC_PARAMS = dataclasses.replace(_SC_PARAMS, needs_layout_passes=False)
if "use_tc_tiling_on_sc" in pltpu.CompilerParams.__dataclass_fields__:
    _SC_PARAMS = dataclasses.replace(_SC_PARAMS, use_tc_tiling_on_sc=False)


def _zero_vec16():
    return jnp.zeros((16,), jnp.float32)


@functools.partial(
    pl.kernel,
    out_type=jax.ShapeDtypeStruct((NW, NP), jnp.float32),
    mesh=_MESH,
    compiler_params=_SC_PARAMS,
    scratch_types=[
        pltpu.VMEM((NCH, CH), jnp.int32),
        pltpu.VMEM((NCH, CH), jnp.float32),
        pltpu.VMEM((NP,), jnp.float32),
    ],
)
def _sc_deg(dst_hbm, w_hbm, out_hbm, dst_v, w_v, acc_v):
    cid = lax.axis_index("c")
    sid = lax.axis_index("s")
    wid = cid * NS + sid

    @pl.loop(0, NP // 16)
    def _(i):
        acc_v[pl.ds(i * 16, 16)] = _zero_vec16()

    pltpu.sync_copy(dst_hbm.at[wid], dst_v)
    pltpu.sync_copy(w_hbm.at[wid], w_v)

    @pl.loop(0, NCH)
    def _(c):
        for j in range(CH // 16):
            idx = dst_v[c, pl.ds(j * 16, 16)]
            val = w_v[c, pl.ds(j * 16, 16)]
            plsc.addupdate_scatter(acc_v, [idx], val)

    pltpu.sync_copy(acc_v, out_hbm.at[wid])


def _make_conv(dim):
    nbuf = 4 if dim <= 64 else 1
    scratch = [
        pltpu.VMEM((NCH, CH), jnp.int32),
        pltpu.VMEM((NCH, CH), jnp.int32),
        pltpu.VMEM((NCH, CH), jnp.float32),
    ]
    scratch += [pltpu.VMEM((CH, dim), jnp.float32) for _ in range(nbuf)]
    scratch += [pltpu.VMEM_SHARED((NP, dim), jnp.float32)]
    scratch += [pltpu.SemaphoreType.DMA for _ in range(2 * nbuf)]

    @functools.partial(
        pl.kernel,
        out_type=jax.ShapeDtypeStruct((2, NP, dim), jnp.float32),
        mesh=_MESH,
        compiler_params=_SC_PARAMS,
        scratch_types=scratch,
    )
    def conv(h_hbm, src_hbm, dst_hbm, w_hbm, out_hbm, src_v, dst_v, w_v,
             *rest):
        bufs = rest[:nbuf]
        acc_sh = rest[nbuf]
        gsems = rest[nbuf + 1:2 * nbuf + 1]
        ssems = rest[2 * nbuf + 1:]
        cid = lax.axis_index("c")
        sid = lax.axis_index("s")
        wid = cid * NS + sid

        @pl.loop(0, CH)
        def _(r):
            for j in range(dim // 16):
                bufs[0][r, pl.ds(j * 16, 16)] = _zero_vec16()

        for k in range(RPS // CH):
            pltpu.sync_copy(bufs[0], acc_sh.at[pl.ds(sid * RPS + k * CH, CH)])
        plsc.subcore_barrier()

        pltpu.sync_copy(src_hbm.at[wid], src_v)
        pltpu.sync_copy(dst_hbm.at[wid], dst_v)
        pltpu.sync_copy(w_hbm.at[wid], w_v)

        def scale(rows_v, c):
            @pl.loop(0, CH // 16)
            def _(g):
                wv = w_v[c, pl.ds(g * 16, 16)]
                for l in range(16):
                    sv = lax.broadcast(wv[l], (16,))
                    for j in range(dim // 16):
                        sl = pl.ds(j * 16, 16)
                        e = g * 16 + l
                        rows_v[e, sl] = rows_v[e, sl] * sv

        if nbuf == 4:
            for b in range(4):
                pltpu.async_copy(h_hbm.at[src_v.at[b]], bufs[b], gsems[b])

            @pl.loop(0, NCH, step=4)
            def _(c):
                for k in range(4):
                    cc = c + k
                    kp = (k - 1) % 4
                    pltpu.make_async_copy(
                        h_hbm.at[src_v.at[cc]], bufs[k], gsems[k]).wait()
                    scale(bufs[k], cc)
                    pltpu.async_copy(
                        bufs[k], acc_sh.at[dst_v.at[cc]], ssems[k], add=True)

                    @pl.when((cc >= 1) & (cc + 3 < NCH))
                    def _(kp=kp, cc=cc):
                        pltpu.make_async_copy(
                            bufs[kp], acc_sh.at[dst_v.at[cc - 1]],
                            ssems[kp]).wait()
                        pltpu.async_copy(
                            h_hbm.at[src_v.at[cc + 3]], bufs[kp], gsems[kp])

            for k in range(4):
                pltpu.make_async_copy(
                    bufs[k], acc_sh.at[dst_v.at[NCH - 4 + k]],
                    ssems[k]).wait()
        else:
            @pl.loop(0, NCH)
            def _(c):
                pltpu.async_copy(h_hbm.at[src_v.at[c]], bufs[0], gsems[0]).wait()
                scale(bufs[0], c)
                pltpu.sync_copy(bufs[0], acc_sh.at[dst_v.at[c]], add=True)

        plsc.subcore_barrier()
        for k in range(RPS // CH):
            sl = pl.ds(sid * RPS + k * CH, CH)
            pltpu.sync_copy(acc_sh.at[sl], out_hbm.at[cid].at[sl])

    return conv


_sc_conv64 = _make_conv(64)
_sc_conv128 = _make_conv(128)


@functools.partial(
    pl.kernel,
    out_type=(
        jax.ShapeDtypeStruct((NW, NCH, CH), jnp.float32),
        jax.ShapeDtypeStruct((NW, NP), jnp.float32),
    ),
    mesh=_MESH,
    compiler_params=_SC_PARAMS,
    scratch_types=[
        pltpu.VMEM((NCH, CH), jnp.int32),
        pltpu.VMEM((NCH, CH), jnp.int32),
        pltpu.VMEM((NCH, CH), jnp.float32),
        pltpu.VMEM((CH, F // 2), jnp.int32),
        pltpu.VMEM((CH, F // 2), jnp.int32),
        pltpu.VMEM((CH, F // 2), jnp.int32),
        pltpu.VMEM((CH, F // 2), jnp.int32),
        pltpu.VMEM((NP,), jnp.float32),
        pltpu.SemaphoreType.DMA,
        pltpu.SemaphoreType.DMA,
        pltpu.SemaphoreType.DMA,
        pltpu.SemaphoreType.DMA,
    ],
)
def _sc_decode(z_hbm, src_hbm, dst_hbm, probs_hbm, degp_hbm,
               src_v, dst_v, probs_v, zs0_v, zd0_v, zs1_v, zd1_v,
               acc_v, ss0, sd0, ss1, sd1):
    cid = lax.axis_index("c")
    sid = lax.axis_index("s")
    wid = cid * NS + sid

    @pl.loop(0, NP // 16)
    def _(i):
        acc_v[pl.ds(i * 16, 16)] = _zero_vec16()

    pltpu.sync_copy(src_hbm.at[wid], src_v)
    pltpu.sync_copy(dst_hbm.at[wid], dst_v)

    lane = lax.iota(jnp.int32, 16)
    ebase = wid * EPW

    perms = {m: lane ^ m for m in (1, 2, 4, 8)}
    masks = {m: (lane & m) == 0 for m in (1, 2, 4, 8)}
    _dn = lax.GatherDimensionNumbers(
        offset_dims=(), collapsed_slice_dims=(0,), start_index_map=(0,))

    def _perm(v, idx):
        return lax.gather(v, idx[:, None], _dn, (1,),
                          mode=lax.GatherScatterMode.PROMISE_IN_BOUNDS)

    def dots_chunk(zs_v, zd_v, c):
        @pl.loop(0, CH // 16)
        def _(g):
            ps = []
            hi = jnp.full((16,), -65536, jnp.int32)
            for l in range(16):
                e = g * 16 + l
                prod = None
                for j in range(F // 32):
                    sl = pl.ds(j * 16, 16)
                    vs = zs_v[e, sl]
                    vd = zd_v[e, sl]
                    se = plsc.bitcast(lax.shift_left(vs, 16), jnp.float32)
                    so = plsc.bitcast(vs & hi, jnp.float32)
                    de = plsc.bitcast(lax.shift_left(vd, 16), jnp.float32)
                    do = plsc.bitcast(vd & hi, jnp.float32)
                    t = se * de + so * do
                    prod = t if prod is None else prod + t
                ps.append(prod)
            for m in (1, 2, 4, 8):
                nxt = []
                for i in range(0, len(ps), 2):
                    a2 = ps[i] + _perm(ps[i], perms[m])
                    b2 = ps[i + 1] + _perm(ps[i + 1], perms[m])
                    nxt.append(jnp.where(masks[m], a2, b2))
                ps = nxt
            dots = ps[0]
            sl = pl.ds(g * 16, 16)
            eid = lane + (ebase + c * CH + g * 16)
            p = jnp.where(eid < E, 1.0 / (1.0 + jnp.exp(-dots)), 0.0)
            probs_v[c, sl] = p
            plsc.addupdate_scatter(acc_v, [dst_v[c, sl]], p)

    pltpu.async_copy(z_hbm.at[src_v.at[0]], zs0_v, ss0)
    pltpu.async_copy(z_hbm.at[dst_v.at[0]], zd0_v, sd0)
    pltpu.async_copy(z_hbm.at[src_v.at[1]], zs1_v, ss1)
    pltpu.async_copy(z_hbm.at[dst_v.at[1]], zd1_v, sd1)

    @pl.loop(0, NCH, step=2)
    def _(c):
        pltpu.make_async_copy(z_hbm.at[src_v.at[c]], zs0_v, ss0).wait()
        pltpu.make_async_copy(z_hbm.at[dst_v.at[c]], zd0_v, sd0).wait()
        dots_chunk(zs0_v, zd0_v, c)

        @pl.when(c + 2 < NCH)
        def _():
            pltpu.async_copy(z_hbm.at[src_v.at[c + 2]], zs0_v, ss0)
            pltpu.async_copy(z_hbm.at[dst_v.at[c + 2]], zd0_v, sd0)

        pltpu.make_async_copy(z_hbm.at[src_v.at[c + 1]], zs1_v, ss1).wait()
        pltpu.make_async_copy(z_hbm.at[dst_v.at[c + 1]], zd1_v, sd1).wait()
        dots_chunk(zs1_v, zd1_v, c + 1)

        @pl.when(c + 3 < NCH)
        def _():
            pltpu.async_copy(z_hbm.at[src_v.at[c + 3]], zs1_v, ss1)
            pltpu.async_copy(z_hbm.at[dst_v.at[c + 3]], zd1_v, sd1)

    pltpu.sync_copy(probs_v, probs_hbm.at[wid])
    pltpu.sync_copy(acc_v, degp_hbm.at[wid])


_BT = 512


def _mm(x, W):
    M, K = x.shape
    _, Nn = W.shape

    def body(x_ref, w_ref, o_ref):
        o_ref[...] = jnp.dot(x_ref[...], w_ref[...],
                             preferred_element_type=jnp.float32)

    return pl.pallas_call(
        body,
        grid=(M // _BT,),
        in_specs=[pl.BlockSpec((_BT, K), lambda i: (i, 0)),
                  pl.BlockSpec((K, Nn), lambda i: (0, 0))],
        out_specs=pl.BlockSpec((_BT, Nn), lambda i: (i, 0)),
        out_shape=jax.ShapeDtypeStruct((M, Nn), jnp.float32),
    )(x, W)


def _norms(degp, h1):
    def body(dp_ref, h1_ref, h1p_ref, dis_ref, invd_ref):
        deg = jnp.sum(dp_ref[...], axis=0, keepdims=True) + 1.0
        dis = lax.rsqrt(deg)
        invd = 1.0 / deg
        dis_ref[...] = dis.T
        invd_ref[...] = invd.T
        h1p_ref[...] = h1_ref[...] * dis.T

    return pl.pallas_call(
        body,
        grid=(NP // _BT,),
        in_specs=[pl.BlockSpec((NW, _BT), lambda i: (0, i)),
                  pl.BlockSpec((_BT, H), lambda i: (i, 0))],
        out_specs=[pl.BlockSpec((_BT, H), lambda i: (i, 0)),
                   pl.BlockSpec((_BT, 1), lambda i: (i, 0)),
                   pl.BlockSpec((_BT, 1), lambda i: (i, 0))],
        out_shape=[jax.ShapeDtypeStruct((NP, H), jnp.float32),
                   jax.ShapeDtypeStruct((NP, 1), jnp.float32),
                   jax.ShapeDtypeStruct((NP, 1), jnp.float32)],
    )(degp, h1)


def _post1(acc, h1, dis, invd, b1, W2):
    def body(acc_ref, h1_ref, dis_ref, invd_ref, b1_ref, w2_ref,
             h2p_ref, h2i_ref):
        s = acc_ref[0] + acc_ref[1]
        h = jnp.maximum(
            dis_ref[...] * s + h1_ref[...] * invd_ref[...] + b1_ref[...], 0.0)
        h2 = jnp.dot(h, w2_ref[...], preferred_element_type=jnp.float32)
        h2p_ref[...] = h2 * dis_ref[...]
        h2i_ref[...] = h2 * invd_ref[...]

    return pl.pallas_call(
        body,
        grid=(NP // _BT,),
        in_specs=[pl.BlockSpec((2, _BT, H), lambda i: (0, i, 0)),
                  pl.BlockSpec((_BT, H), lambda i: (i, 0)),
                  pl.BlockSpec((_BT, 1), lambda i: (i, 0)),
                  pl.BlockSpec((_BT, 1), lambda i: (i, 0)),
                  pl.BlockSpec((1, H), lambda i: (0, 0)),
                  pl.BlockSpec((H, F), lambda i: (0, 0))],
        out_specs=[pl.BlockSpec((_BT, F), lambda i: (i, 0)),
                   pl.BlockSpec((_BT, F), lambda i: (i, 0))],
        out_shape=[jax.ShapeDtypeStruct((NP, F), jnp.float32),
                   jax.ShapeDtypeStruct((NP, F), jnp.float32)],
    )(acc, h1, dis, invd, b1, W2)


def _post2(acc, h2i, dis, b2, x, Wd):
    def body(acc_ref, h2i_ref, dis_ref, b2_ref, x_ref, wd_ref, z_ref, y_ref):
        s = acc_ref[0] + acc_ref[1]
        logits = dis_ref[...] * s + h2i_ref[...] + b2_ref[...]
        z = 1.0 / (1.0 + jnp.exp(-logits))
        z_ref[...] = z
        y_ref[...] = jnp.dot(z + x_ref[...], wd_ref[...],
                             preferred_element_type=jnp.float32)

    return pl.pallas_call(
        body,
        grid=(NP // _BT,),
        in_specs=[pl.BlockSpec((2, _BT, F), lambda i: (0, i, 0)),
                  pl.BlockSpec((_BT, F), lambda i: (i, 0)),
                  pl.BlockSpec((_BT, 1), lambda i: (i, 0)),
                  pl.BlockSpec((1, F), lambda i: (0, 0)),
                  pl.BlockSpec((_BT, F), lambda i: (i, 0)),
                  pl.BlockSpec((F, H), lambda i: (0, 0))],
        out_specs=[pl.BlockSpec((_BT, F), lambda i: (i, 0)),
                   pl.BlockSpec((_BT, H), lambda i: (i, 0))],
        out_shape=[jax.ShapeDtypeStruct((NP, F), jnp.float32),
                   jax.ShapeDtypeStruct((NP, H), jnp.float32)],
    )(acc, h2i, dis, b2, x, Wd)


def _norms2(degp, y):
    def body(dp_ref, y_ref, yp_ref, yi_ref, dis_ref):
        deg = jnp.sum(dp_ref[...], axis=0, keepdims=True) + 1.0
        dis = lax.rsqrt(deg)
        invd = 1.0 / deg
        yp_ref[...] = y_ref[...] * dis.T
        yi_ref[...] = y_ref[...] * invd.T
        dis_ref[...] = dis.T

    return pl.pallas_call(
        body,
        grid=(NP // _BT,),
        in_specs=[pl.BlockSpec((NW, _BT), lambda i: (0, i)),
                  pl.BlockSpec((_BT, H), lambda i: (i, 0))],
        out_specs=[pl.BlockSpec((_BT, H), lambda i: (i, 0)),
                   pl.BlockSpec((_BT, H), lambda i: (i, 0)),
                   pl.BlockSpec((_BT, 1), lambda i: (i, 0))],
        out_shape=[jax.ShapeDtypeStruct((NP, H), jnp.float32),
                   jax.ShapeDtypeStruct((NP, H), jnp.float32),
                   jax.ShapeDtypeStruct((NP, 1), jnp.float32)],
    )(degp, y)


def _head(acc, yi, dis2, bd, Wfc2d, bfc):
    nsteps = NP // _BT

    def body(acc_ref, yi_ref, dis_ref, bd_ref, wfc_ref, bfc_ref, o_ref):
        i = pl.program_id(0)

        @pl.when(i == 0)
        def _():
            o_ref[...] = jnp.zeros((1, 1), jnp.float32)

        s = acc_ref[0] + acc_ref[1]
        hd = jnp.maximum(
            dis_ref[...] * s + yi_ref[...] + bd_ref[...], 0.0)
        o_ref[...] += jnp.sum(hd * wfc_ref[...]).reshape(1, 1)

        @pl.when(i == nsteps - 1)
        def _():
            t = o_ref[...] + bfc_ref[...]
            o_ref[...] = 1.0 / (1.0 + jnp.exp(-t))

    return pl.pallas_call(
        body,
        grid=(nsteps,),
        in_specs=[pl.BlockSpec((2, _BT, H), lambda i: (0, i, 0)),
                  pl.BlockSpec((_BT, H), lambda i: (i, 0)),
                  pl.BlockSpec((_BT, 1), lambda i: (i, 0)),
                  pl.BlockSpec((1, H), lambda i: (0, 0)),
                  pl.BlockSpec((_BT, H), lambda i: (i, 0)),
                  pl.BlockSpec((1, 1), lambda i: (0, 0))],
        out_specs=pl.BlockSpec((1, 1), lambda i: (0, 0)),
        out_shape=jax.ShapeDtypeStruct((1, 1), jnp.float32),
    )(acc, yi, dis2, bd, Wfc2d, bfc)


def kernel(features, edge_index, edge_attr, W1, b1, W2, b2, Wd, bd, Wfc, bfc):
    pad = EP - E
    src = jnp.concatenate(
        [edge_index[0].astype(jnp.int32), jnp.zeros((pad,), jnp.int32)]
    ).reshape(NW, NCH, CH)
    dst = jnp.concatenate(
        [edge_index[1].astype(jnp.int32), jnp.zeros((pad,), jnp.int32)]
    ).reshape(NW, NCH, CH)
    w = jnp.concatenate(
        [edge_attr.astype(jnp.float32), jnp.zeros((pad,), jnp.float32)]
    ).reshape(NW, NCH, CH)
    xP = jnp.pad(features, ((0, NP - N), (0, 0)))
    WfcP = jnp.pad(Wfc.reshape(N, H), ((0, NP - N), (0, 0)))
    b1r = b1.reshape(1, H)
    b2r = b2.reshape(1, F)
    bdr = bd.reshape(1, H)
    bfcr = bfc.reshape(1, 1)

    degp = _sc_deg(dst, w)
    h1 = _mm(xP, W1)
    h1p, dis, invd = _norms(degp, h1)
    acc1 = _sc_conv64(h1p, src, dst, w)
    h2p, h2i = _post1(acc1, h1, dis, invd, b1r, W2)
    acc2 = _sc_conv128(h2p, src, dst, w)
    z, y = _post2(acc2, h2i, dis, b2r, xP, Wd)
    z_bf = z.astype(jnp.bfloat16).reshape(NP, F // 2, 2)
    z_i32 = lax.bitcast_convert_type(z_bf, jnp.int32)
    probs, degp2 = _sc_decode(z_i32, src, dst)
    yp, yi, dis2 = _norms2(degp2, y)
    acc3 = _sc_conv64(yp, src, dst, probs)
    out = _head(acc3, yi, dis2, bdr, WfcP, bfcr)
    return out[0, 0]


# speedup vs baseline: 1.3584x; 1.0122x over previous
import dataclasses
import functools

import numpy as np

import jax
import jax.numpy as jnp
from jax import lax
from jax.experimental import pallas as pl
from jax.experimental.pallas import tpu as pltpu
from jax.experimental.pallas import tpu_sc as plsc

N = 10000
NP = 10240
F = 128
H = 64
E = 320000
NS = 16
NW = 32
EPW = 10240
CH = 128
NCH = EPW // CH
EP = NW * EPW
RPS = NP // NS

_MESH = plsc.VectorSubcoreMesh(
    core_axis_name="c", subcore_axis_name="s", num_cores=2, num_subcores=16
)

_SC_PARAMS = pltpu.CompilerParams()
if "needs_layout_passes" in pltpu.CompilerParams.__dataclass_fields__:
    _SC_PARAMS = dataclasses.replace(_SC_PARAMS, needs_layout_passes=False)
if "use_tc_tiling_on_sc" in pltpu.CompilerParams.__dataclass_fields__:
    _SC_PARAMS = dataclasses.replace(_SC_PARAMS, use_tc_tiling_on_sc=False)


def _unpack_perm(dim):
    out = []
    for b in range(dim // 32):
        out.extend(range(32 * b, 32 * b + 32, 2))
        out.extend(range(32 * b + 1, 32 * b + 32, 2))
    return np.array(out)


_U64 = _unpack_perm(64)


def _zero_vec16():
    return jnp.zeros((16,), jnp.float32)


def _pack_bf16(x_bf):
    m, d = x_bf.shape
    return lax.bitcast_convert_type(x_bf.reshape(m, d // 2, 2), jnp.int32)


@functools.partial(
    pl.kernel,
    out_type=jax.ShapeDtypeStruct((NW, NP), jnp.float32),
    mesh=_MESH,
    compiler_params=_SC_PARAMS,
    scratch_types=[
        pltpu.VMEM((NCH, CH), jnp.int32),
        pltpu.VMEM((NCH, CH), jnp.float32),
        pltpu.VMEM((NP,), jnp.float32),
    ],
)
def _sc_deg(dst_hbm, w_hbm, out_hbm, dst_v, w_v, acc_v):
    cid = lax.axis_index("c")
    sid = lax.axis_index("s")
    wid = cid * NS + sid

    @pl.loop(0, NP // 16)
    def _(i):
        acc_v[pl.ds(i * 16, 16)] = _zero_vec16()

    pltpu.sync_copy(dst_hbm.at[wid], dst_v)
    pltpu.sync_copy(w_hbm.at[wid], w_v)

    @pl.loop(0, NCH)
    def _(c):
        for j in range(CH // 16):
            idx = dst_v[c, pl.ds(j * 16, 16)]
            val = w_v[c, pl.ds(j * 16, 16)]
            plsc.addupdate_scatter(acc_v, [idx], val)

    pltpu.sync_copy(acc_v, out_hbm.at[wid])


def _make_conv(dim, packed):
    nbuf = 4 if packed else 1
    gdim = dim // 2 if packed else dim
    gdt = jnp.int32 if packed else jnp.float32
    scratch = [
        pltpu.VMEM((NCH, CH), jnp.int32),
        pltpu.VMEM((NCH, CH), jnp.int32),
        pltpu.VMEM((NCH, CH), jnp.float32),
    ]
    scratch += [pltpu.VMEM((CH, gdim), gdt) for _ in range(nbuf)]
    if packed:
        scratch += [pltpu.VMEM((CH, dim), jnp.float32) for _ in range(nbuf)]
    scratch += [pltpu.VMEM_SHARED((NP, dim), jnp.float32)]
    scratch += [pltpu.SemaphoreType.DMA for _ in range(2 * nbuf)]

    @functools.partial(
        pl.kernel,
        out_type=jax.ShapeDtypeStruct((2, NP, dim), jnp.float32),
        mesh=_MESH,
        compiler_params=_SC_PARAMS,
        scratch_types=scratch,
    )
    def conv(h_hbm, src_hbm, dst_hbm, w_hbm, out_hbm, src_v, dst_v, w_v,
             *rest):
        gbufs = rest[:nbuf]
        rest = rest[nbuf:]
        if packed:
            fbufs = rest[:nbuf]
            rest = rest[nbuf:]
        else:
            fbufs = gbufs
        acc_sh = rest[0]
        gsems = rest[1:nbuf + 1]
        ssems = rest[nbuf + 1:]
        cid = lax.axis_index("c")
        sid = lax.axis_index("s")
        wid = cid * NS + sid

        @pl.loop(0, CH)
        def _(r):
            for j in range(dim // 16):
                fbufs[0][r, pl.ds(j * 16, 16)] = _zero_vec16()

        for k in range(RPS // CH):
            pltpu.sync_copy(fbufs[0], acc_sh.at[pl.ds(sid * RPS + k * CH, CH)])
        plsc.subcore_barrier()

        pltpu.sync_copy(src_hbm.at[wid], src_v)
        pltpu.sync_copy(dst_hbm.at[wid], dst_v)
        pltpu.sync_copy(w_hbm.at[wid], w_v)

        hi = jnp.full((16,), -65536, jnp.int32)

        def scale(rows_i, rows_f, c):
            @pl.loop(0, CH // 16)
            def _(g):
                wv = w_v[c, pl.ds(g * 16, 16)]
                for l in range(16):
                    sv = lax.broadcast(wv[l], (16,))
                    e = g * 16 + l
                    if packed:
                        for j in range(dim // 32):
                            v = rows_i[e, pl.ds(j * 16, 16)]
                            fe = plsc.bitcast(lax.shift_left(v, 16),
                                              jnp.float32)
                            fo = plsc.bitcast(v & hi, jnp.float32)
                            rows_f[e, pl.ds((2 * j) * 16, 16)] = fe * sv
                            rows_f[e, pl.ds((2 * j + 1) * 16, 16)] = fo * sv
                    else:
                        for j in range(dim // 16):
                            sl = pl.ds(j * 16, 16)
                            rows_f[e, sl] = rows_f[e, sl] * sv

        if packed:
            for b in range(nbuf):
                pltpu.async_copy(h_hbm.at[src_v.at[b]], gbufs[b], gsems[b])

            @pl.loop(0, NCH, step=nbuf)
            def _(c):
                for k in range(nbuf):
                    cc = c + k
                    pltpu.make_async_copy(
                        h_hbm.at[src_v.at[cc]], gbufs[k], gsems[k]).wait()

                    @pl.when(cc >= nbuf)
                    def _(k=k, cc=cc):
                        pltpu.make_async_copy(
                            fbufs[k], acc_sh.at[dst_v.at[cc - nbuf]],
                            ssems[k]).wait()

                    scale(gbufs[k], fbufs[k], cc)
                    pltpu.async_copy(
                        fbufs[k], acc_sh.at[dst_v.at[cc]], ssems[k], add=True)

                    @pl.when(cc + nbuf < NCH)
                    def _(k=k, cc=cc):
                        pltpu.async_copy(
                            h_hbm.at[src_v.at[cc + nbuf]], gbufs[k], gsems[k])

            for k in range(nbuf):
                pltpu.make_async_copy(
                    fbufs[k], acc_sh.at[dst_v.at[NCH - nbuf + k]],
                    ssems[k]).wait()
        else:
            @pl.loop(0, NCH)
            def _(c):
                pltpu.async_copy(h_hbm.at[src_v.at[c]], gbufs[0],
                                 gsems[0]).wait()
                scale(gbufs[0], fbufs[0], c)
                pltpu.sync_copy(fbufs[0], acc_sh.at[dst_v.at[c]], add=True)

        plsc.subcore_barrier()
        for k in range(RPS // CH):
            sl = pl.ds(sid * RPS + k * CH, CH)
            pltpu.sync_copy(acc_sh.at[sl], out_hbm.at[cid].at[sl])

    return conv


_sc_conv64 = _make_conv(64, packed=True)
_sc_conv128 = _make_conv(128, packed=False)


@functools.partial(
    pl.kernel,
    out_type=(
        jax.ShapeDtypeStruct((NW, NCH, CH), jnp.float32),
        jax.ShapeDtypeStruct((NW, NP), jnp.float32),
    ),
    mesh=_MESH,
    compiler_params=_SC_PARAMS,
    scratch_types=[
        pltpu.VMEM((NCH, CH), jnp.int32),
        pltpu.VMEM((NCH, CH), jnp.int32),
        pltpu.VMEM((NCH, CH), jnp.float32),
        pltpu.VMEM((CH, F // 2), jnp.int32),
        pltpu.VMEM((CH, F // 2), jnp.int32),
        pltpu.VMEM((CH, F // 2), jnp.int32),
        pltpu.VMEM((CH, F // 2), jnp.int32),
        pltpu.VMEM((NP,), jnp.float32),
        pltpu.SemaphoreType.DMA,
        pltpu.SemaphoreType.DMA,
        pltpu.SemaphoreType.DMA,
        pltpu.SemaphoreType.DMA,
    ],
)
def _sc_decode(z_hbm, src_hbm, dst_hbm, probs_hbm, degp_hbm,
               src_v, dst_v, probs_v, zs0_v, zd0_v, zs1_v, zd1_v,
               acc_v, ss0, sd0, ss1, sd1):
    cid = lax.axis_index("c")
    sid = lax.axis_index("s")
    wid = cid * NS + sid

    @pl.loop(0, NP // 16)
    def _(i):
        acc_v[pl.ds(i * 16, 16)] = _zero_vec16()

    pltpu.sync_copy(src_hbm.at[wid], src_v)
    pltpu.sync_copy(dst_hbm.at[wid], dst_v)

    lane = lax.iota(jnp.int32, 16)
    ebase = wid * EPW
    perms = {m: lane ^ m for m in (1, 2, 4, 8)}
    masks = {m: (lane & m) == 0 for m in (1, 2, 4, 8)}
    _dn = lax.GatherDimensionNumbers(
        offset_dims=(), collapsed_slice_dims=(0,), start_index_map=(0,))

    def _perm(v, idx):
        return lax.gather(v, idx[:, None], _dn, (1,),
                          mode=lax.GatherScatterMode.PROMISE_IN_BOUNDS)

    def dots_chunk(zs_v, zd_v, c):
        @pl.loop(0, CH // 16)
        def _(g):
            ps = []
            hi = jnp.full((16,), -65536, jnp.int32)
            for l in range(16):
                e = g * 16 + l
                prod = None
                for j in range(F // 32):
                    sl = pl.ds(j * 16, 16)
                    vs = zs_v[e, sl]
                    vd = zd_v[e, sl]
                    se = plsc.bitcast(lax.shift_left(vs, 16), jnp.float32)
                    so = plsc.bitcast(vs & hi, jnp.float32)
                    de = plsc.bitcast(lax.shift_left(vd, 16), jnp.float32)
                    do = plsc.bitcast(vd & hi, jnp.float32)
                    t = se * de + so * do
                    prod = t if prod is None else prod + t
                ps.append(prod)
            for m in (1, 2, 4, 8):
                nxt = []
                for i in range(0, len(ps), 2):
                    a2 = ps[i] + _perm(ps[i], perms[m])
                    b2 = ps[i + 1] + _perm(ps[i + 1], perms[m])
                    nxt.append(jnp.where(masks[m], a2, b2))
                ps = nxt
            dots = ps[0]
            sl = pl.ds(g * 16, 16)
            eid = lane + (ebase + c * CH + g * 16)
            p = jnp.where(eid < E, 1.0 / (1.0 + jnp.exp(-dots)), 0.0)
            probs_v[c, sl] = p
            plsc.addupdate_scatter(acc_v, [dst_v[c, sl]], p)

    pltpu.async_copy(z_hbm.at[src_v.at[0]], zs0_v, ss0)
    pltpu.async_copy(z_hbm.at[dst_v.at[0]], zd0_v, sd0)
    pltpu.async_copy(z_hbm.at[src_v.at[1]], zs1_v, ss1)
    pltpu.async_copy(z_hbm.at[dst_v.at[1]], zd1_v, sd1)

    @pl.loop(0, NCH, step=2)
    def _(c):
        pltpu.make_async_copy(z_hbm.at[src_v.at[c]], zs0_v, ss0).wait()
        pltpu.make_async_copy(z_hbm.at[dst_v.at[c]], zd0_v, sd0).wait()
        dots_chunk(zs0_v, zd0_v, c)

        @pl.when(c + 2 < NCH)
        def _():
            pltpu.async_copy(z_hbm.at[src_v.at[c + 2]], zs0_v, ss0)
            pltpu.async_copy(z_hbm.at[dst_v.at[c + 2]], zd0_v, sd0)

        pltpu.make_async_copy(z_hbm.at[src_v.at[c + 1]], zs1_v, ss1).wait()
        pltpu.make_async_copy(z_hbm.at[dst_v.at[c + 1]], zd1_v, sd1).wait()
        dots_chunk(zs1_v, zd1_v, c + 1)

        @pl.when(c + 3 < NCH)
        def _():
            pltpu.async_copy(z_hbm.at[src_v.at[c + 3]], zs1_v, ss1)
            pltpu.async_copy(z_hbm.at[dst_v.at[c + 3]], zd1_v, sd1)

    pltpu.sync_copy(probs_v, probs_hbm.at[wid])
    pltpu.sync_copy(acc_v, degp_hbm.at[wid])


_BT = 512


def _mm2(x, Wn, Wa):
    M, K = x.shape
    _, Nn = Wn.shape

    def body(x_ref, wn_ref, wa_ref, on_ref, oa_ref):
        xb = x_ref[...]
        on_ref[...] = jnp.dot(xb, wn_ref[...],
                              preferred_element_type=jnp.float32)
        oa_ref[...] = jnp.dot(xb, wa_ref[...],
                              preferred_element_type=jnp.float32)

    return pl.pallas_call(
        body,
        grid=(M // _BT,),
        in_specs=[pl.BlockSpec((_BT, K), lambda i: (i, 0)),
                  pl.BlockSpec((K, Nn), lambda i: (0, 0)),
                  pl.BlockSpec((K, Nn), lambda i: (0, 0))],
        out_specs=[pl.BlockSpec((_BT, Nn), lambda i: (i, 0)),
                   pl.BlockSpec((_BT, Nn), lambda i: (i, 0))],
        out_shape=[jax.ShapeDtypeStruct((M, Nn), jnp.float32),
                   jax.ShapeDtypeStruct((M, Nn), jnp.float32)],
    )(x, Wn, Wa)


def _norms(degp, h1n, h1a):
    def body(dp_ref, h1n_ref, h1a_ref, h1p_ref, h1ai_ref, dis_ref, invd_ref):
        deg = jnp.sum(dp_ref[...], axis=0, keepdims=True) + 1.0
        dis = lax.rsqrt(deg)
        invd = 1.0 / deg
        dis_ref[...] = dis.T
        invd_ref[...] = invd.T
        h1p_ref[...] = (h1n_ref[...] * dis.T).astype(jnp.bfloat16)
        h1ai_ref[...] = h1a_ref[...] * invd.T

    return pl.pallas_call(
        body,
        grid=(NP // _BT,),
        in_specs=[pl.BlockSpec((NW, _BT), lambda i: (0, i)),
                  pl.BlockSpec((_BT, H), lambda i: (i, 0)),
                  pl.BlockSpec((_BT, H), lambda i: (i, 0))],
        out_specs=[pl.BlockSpec((_BT, H), lambda i: (i, 0)),
                   pl.BlockSpec((_BT, H), lambda i: (i, 0)),
                   pl.BlockSpec((_BT, 1), lambda i: (i, 0)),
                   pl.BlockSpec((_BT, 1), lambda i: (i, 0))],
        out_shape=[jax.ShapeDtypeStruct((NP, H), jnp.bfloat16),
                   jax.ShapeDtypeStruct((NP, H), jnp.float32),
                   jax.ShapeDtypeStruct((NP, 1), jnp.float32),
                   jax.ShapeDtypeStruct((NP, 1), jnp.float32)],
    )(degp, h1n, h1a)


def _post1(acc, h1ai, dis, invd, b1t, W2s):
    def body(acc_ref, h1ai_ref, dis_ref, invd_ref, b1_ref, w2s_ref,
             h2p_ref, h2i_ref):
        s = acc_ref[0] + acc_ref[1]
        h = jnp.maximum(
            dis_ref[...] * s + h1ai_ref[...] + b1_ref[...], 0.0)
        h2 = jnp.dot(h, w2s_ref[...], preferred_element_type=jnp.float32)
        h2p_ref[...] = h2 * dis_ref[...]
        h2i_ref[...] = h2 * invd_ref[...]

    return pl.pallas_call(
        body,
        grid=(NP // _BT,),
        in_specs=[pl.BlockSpec((2, _BT, H), lambda i: (0, i, 0)),
                  pl.BlockSpec((_BT, H), lambda i: (i, 0)),
                  pl.BlockSpec((_BT, 1), lambda i: (i, 0)),
                  pl.BlockSpec((_BT, 1), lambda i: (i, 0)),
                  pl.BlockSpec((1, H), lambda i: (0, 0)),
                  pl.BlockSpec((H, F), lambda i: (0, 0))],
        out_specs=[pl.BlockSpec((_BT, F), lambda i: (i, 0)),
                   pl.BlockSpec((_BT, F), lambda i: (i, 0))],
        out_shape=[jax.ShapeDtypeStruct((NP, F), jnp.float32),
                   jax.ShapeDtypeStruct((NP, F), jnp.float32)],
    )(acc, h1ai, dis, invd, b1t, W2s)


def _post2(acc, h2i, dis, b2, x, Wd, Wda):
    def body(acc_ref, h2i_ref, dis_ref, b2_ref, x_ref, wd_ref, wda_ref,
             z_ref, yn_ref, ya_ref):
        s = acc_ref[0] + acc_ref[1]
        logits = dis_ref[...] * s + h2i_ref[...] + b2_ref[...]
        z = 1.0 / (1.0 + jnp.exp(-logits))
        z_ref[...] = z.astype(jnp.bfloat16)
        zres = z + x_ref[...]
        yn_ref[...] = jnp.dot(zres, wd_ref[...],
                              preferred_element_type=jnp.float32)
        ya_ref[...] = jnp.dot(zres, wda_ref[...],
                              preferred_element_type=jnp.float32)

    return pl.pallas_call(
        body,
        grid=(NP // _BT,),
        in_specs=[pl.BlockSpec((2, _BT, F), lambda i: (0, i, 0)),
                  pl.BlockSpec((_BT, F), lambda i: (i, 0)),
                  pl.BlockSpec((_BT, 1), lambda i: (i, 0)),
                  pl.BlockSpec((1, F), lambda i: (0, 0)),
                  pl.BlockSpec((_BT, F), lambda i: (i, 0)),
                  pl.BlockSpec((F, H), lambda i: (0, 0)),
                  pl.BlockSpec((F, H), lambda i: (0, 0))],
        out_specs=[pl.BlockSpec((_BT, F), lambda i: (i, 0)),
                   pl.BlockSpec((_BT, H), lambda i: (i, 0)),
                   pl.BlockSpec((_BT, H), lambda i: (i, 0))],
        out_shape=[jax.ShapeDtypeStruct((NP, F), jnp.bfloat16),
                   jax.ShapeDtypeStruct((NP, H), jnp.float32),
                   jax.ShapeDtypeStruct((NP, H), jnp.float32)],
    )(acc, h2i, dis, b2, x, Wd, Wda)


def _norms2(degp, yn, ya):
    def body(dp_ref, yn_ref, ya_ref, yp_ref, yai_ref, dis_ref):
        deg = jnp.sum(dp_ref[...], axis=0, keepdims=True) + 1.0
        dis = lax.rsqrt(deg)
        invd = 1.0 / deg
        yp_ref[...] = (yn_ref[...] * dis.T).astype(jnp.bfloat16)
        yai_ref[...] = ya_ref[...] * invd.T
        dis_ref[...] = dis.T

    return pl.pallas_call(
        body,
        grid=(NP // _BT,),
        in_specs=[pl.BlockSpec((NW, _BT), lambda i: (0, i)),
                  pl.BlockSpec((_BT, H), lambda i: (i, 0)),
                  pl.BlockSpec((_BT, H), lambda i: (i, 0))],
        out_specs=[pl.BlockSpec((_BT, H), lambda i: (i, 0)),
                   pl.BlockSpec((_BT, H), lambda i: (i, 0)),
                   pl.BlockSpec((_BT, 1), lambda i: (i, 0))],
        out_shape=[jax.ShapeDtypeStruct((NP, H), jnp.bfloat16),
                   jax.ShapeDtypeStruct((NP, H), jnp.float32),
                   jax.ShapeDtypeStruct((NP, 1), jnp.float32)],
    )(degp, yn, ya)


def _head(acc, yai, dis2, bdt, Wfct, bfc):
    nsteps = NP // _BT

    def body(acc_ref, yai_ref, dis_ref, bd_ref, wfc_ref, bfc_ref, o_ref):
        i = pl.program_id(0)

        @pl.when(i == 0)
        def _():
            o_ref[...] = jnp.zeros((1, 1), jnp.float32)

        s = acc_ref[0] + acc_ref[1]
        hd = jnp.maximum(
            dis_ref[...] * s + yai_ref[...] + bd_ref[...], 0.0)
        o_ref[...] += jnp.sum(hd * wfc_ref[...]).reshape(1, 1)

        @pl.when(i == nsteps - 1)
        def _():
            t = o_ref[...] + bfc_ref[...]
            o_ref[...] = 1.0 / (1.0 + jnp.exp(-t))

    return pl.pallas_call(
        body,
        grid=(nsteps,),
        in_specs=[pl.BlockSpec((2, _BT, H), lambda i: (0, i, 0)),
                  pl.BlockSpec((_BT, H), lambda i: (i, 0)),
                  pl.BlockSpec((_BT, 1), lambda i: (i, 0)),
                  pl.BlockSpec((1, H), lambda i: (0, 0)),
                  pl.BlockSpec((_BT, H), lambda i: (i, 0)),
                  pl.BlockSpec((1, 1), lambda i: (0, 0))],
        out_specs=pl.BlockSpec((1, 1), lambda i: (0, 0)),
        out_shape=jax.ShapeDtypeStruct((1, 1), jnp.float32),
    )(acc, yai, dis2, bdt, Wfct, bfc)


def kernel(features, edge_index, edge_attr, W1, b1, W2, b2, Wd, bd, Wfc, bfc):
    pad = EP - E
    src = jnp.concatenate(
        [edge_index[0].astype(jnp.int32), jnp.zeros((pad,), jnp.int32)]
    ).reshape(NW, NCH, CH)
    dst = jnp.concatenate(
        [edge_index[1].astype(jnp.int32), jnp.zeros((pad,), jnp.int32)]
    ).reshape(NW, NCH, CH)
    w = jnp.concatenate(
        [edge_attr.astype(jnp.float32), jnp.zeros((pad,), jnp.float32)]
    ).reshape(NW, NCH, CH)

    xP = jnp.pad(features, ((0, NP - N), (0, 0)))
    W1a = W1[:, _U64]
    b1t = b1[_U64].reshape(1, H)
    W2s = W2[_U64, :]
    b2r = b2.reshape(1, F)
    Wda = Wd[:, _U64]
    bdt = bd[_U64].reshape(1, H)
    WfcP = jnp.pad(Wfc.reshape(N, H), ((0, NP - N), (0, 0)))
    Wfct = WfcP[:, _U64]
    bfcr = bfc.reshape(1, 1)

    degp = _sc_deg(dst, w)
    h1n, h1a = _mm2(xP, W1, W1a)
    h1p_bf, h1ai, dis, invd = _norms(degp, h1n, h1a)
    acc1 = _sc_conv64(_pack_bf16(h1p_bf), src, dst, w)
    h2p, h2i = _post1(acc1, h1ai, dis, invd, b1t, W2s)
    acc2 = _sc_conv128(h2p, src, dst, w)
    z_bf, yn, ya = _post2(acc2, h2i, dis, b2r, xP, Wd, Wda)
    probs, degp2 = _sc_decode(_pack_bf16(z_bf), src, dst)
    yp_bf, yai, dis2 = _norms2(degp2, yn, ya)
    acc3 = _sc_conv64(_pack_bf16(yp_bf), src, dst, probs)
    out = _head(acc3, yai, dis2, bdt, Wfct, bfcr)
    return out[0, 0]


# speedup vs baseline: 1.7386x; 1.2799x over previous
import dataclasses
import functools

import numpy as np

import jax
import jax.numpy as jnp
from jax import lax
from jax.experimental import pallas as pl
from jax.experimental.pallas import tpu as pltpu
from jax.experimental.pallas import tpu_sc as plsc

N = 10000
NP = 10240
F = 128
H = 64
E = 320000
NS = 16
NW = 32
EPW = 10240
CH = 128
NCH = EPW // CH
EP = NW * EPW
RPS = NP // NS

_MESH = plsc.VectorSubcoreMesh(
    core_axis_name="c", subcore_axis_name="s", num_cores=2, num_subcores=16
)

_SC_PARAMS = pltpu.CompilerParams()
if "needs_layout_passes" in pltpu.CompilerParams.__dataclass_fields__:
    _SC_PARAMS = dataclasses.replace(_SC_PARAMS, needs_layout_passes=False)
if "use_tc_tiling_on_sc" in pltpu.CompilerParams.__dataclass_fields__:
    _SC_PARAMS = dataclasses.replace(_SC_PARAMS, use_tc_tiling_on_sc=False)


def _unpack_perm(dim):
    out = []
    for b in range(dim // 32):
        out.extend(range(32 * b, 32 * b + 32, 2))
        out.extend(range(32 * b + 1, 32 * b + 32, 2))
    return np.array(out)


_U64 = _unpack_perm(64)
_U128 = _unpack_perm(128)


def _zero_vec16():
    return jnp.zeros((16,), jnp.float32)


def _pack_bf16(x_bf):
    m, d = x_bf.shape
    return lax.bitcast_convert_type(x_bf.reshape(m, d // 2, 2), jnp.int32)


@functools.partial(
    pl.kernel,
    out_type=jax.ShapeDtypeStruct((NW, NP), jnp.float32),
    mesh=_MESH,
    compiler_params=_SC_PARAMS,
    scratch_types=[
        pltpu.VMEM((NCH, CH), jnp.int32),
        pltpu.VMEM((NCH, CH), jnp.float32),
        pltpu.VMEM((NP,), jnp.float32),
    ],
)
def _sc_deg(dst_hbm, w_hbm, out_hbm, dst_v, w_v, acc_v):
    cid = lax.axis_index("c")
    sid = lax.axis_index("s")
    wid = cid * NS + sid

    @pl.loop(0, NP // 16)
    def _(i):
        acc_v[pl.ds(i * 16, 16)] = _zero_vec16()

    pltpu.sync_copy(dst_hbm.at[wid], dst_v)
    pltpu.sync_copy(w_hbm.at[wid], w_v)

    @pl.loop(0, NCH)
    def _(c):
        for j in range(CH // 16):
            idx = dst_v[c, pl.ds(j * 16, 16)]
            val = w_v[c, pl.ds(j * 16, 16)]
            plsc.addupdate_scatter(acc_v, [idx], val)

    pltpu.sync_copy(acc_v, out_hbm.at[wid])


def _make_conv(dim, packed):
    assert packed
    nbuf = 4 if dim <= 64 else 2
    nfb = nbuf if dim <= 64 else 1
    gdim = dim // 2
    scratch = [
        pltpu.VMEM((NCH, CH), jnp.int32),
        pltpu.VMEM((NCH, CH), jnp.int32),
        pltpu.VMEM((NCH, CH), jnp.float32),
    ]
    scratch += [pltpu.VMEM((CH, gdim), jnp.int32) for _ in range(nbuf)]
    scratch += [pltpu.VMEM((CH, dim), jnp.float32) for _ in range(nfb)]
    scratch += [pltpu.VMEM_SHARED((NP, dim), jnp.float32)]
    scratch += [pltpu.SemaphoreType.DMA for _ in range(2 * nbuf)]

    @functools.partial(
        pl.kernel,
        out_type=jax.ShapeDtypeStruct((2, NP, dim), jnp.float32),
        mesh=_MESH,
        compiler_params=_SC_PARAMS,
        scratch_types=scratch,
    )
    def conv(h_hbm, src_hbm, dst_hbm, w_hbm, out_hbm, src_v, dst_v, w_v,
             *rest):
        gbufs = rest[:nbuf]
        rest = rest[nbuf:]
        fbufs = rest[:nfb]
        rest = rest[nfb:]
        acc_sh = rest[0]
        gsems = rest[1:nbuf + 1]
        ssems = rest[nbuf + 1:]
        cid = lax.axis_index("c")
        sid = lax.axis_index("s")
        wid = cid * NS + sid

        @pl.loop(0, CH)
        def _(r):
            for j in range(dim // 16):
                fbufs[0][r, pl.ds(j * 16, 16)] = _zero_vec16()

        for k in range(RPS // CH):
            pltpu.sync_copy(fbufs[0], acc_sh.at[pl.ds(sid * RPS + k * CH, CH)])
        plsc.subcore_barrier()

        pltpu.sync_copy(src_hbm.at[wid], src_v)
        pltpu.sync_copy(dst_hbm.at[wid], dst_v)
        pltpu.sync_copy(w_hbm.at[wid], w_v)

        hi = jnp.full((16,), -65536, jnp.int32)

        def scale(rows_i, rows_f, c):
            @pl.loop(0, CH // 16)
            def _(g):
                wv = w_v[c, pl.ds(g * 16, 16)]
                for l in range(16):
                    sv = lax.broadcast(wv[l], (16,))
                    e = g * 16 + l
                    for j in range(dim // 32):
                        v = rows_i[e, pl.ds(j * 16, 16)]
                        fe = plsc.bitcast(lax.shift_left(v, 16),
                                          jnp.float32)
                        fo = plsc.bitcast(v & hi, jnp.float32)
                        rows_f[e, pl.ds((2 * j) * 16, 16)] = fe * sv
                        rows_f[e, pl.ds((2 * j + 1) * 16, 16)] = fo * sv

        if nfb == nbuf:
            for b in range(nbuf):
                pltpu.async_copy(h_hbm.at[src_v.at[b]], gbufs[b], gsems[b])

            @pl.loop(0, NCH, step=nbuf)
            def _(c):
                for k in range(nbuf):
                    cc = c + k
                    pltpu.make_async_copy(
                        h_hbm.at[src_v.at[cc]], gbufs[k], gsems[k]).wait()

                    @pl.when(cc >= nbuf)
                    def _(k=k, cc=cc):
                        pltpu.make_async_copy(
                            fbufs[k], acc_sh.at[dst_v.at[cc - nbuf]],
                            ssems[k]).wait()

                    scale(gbufs[k], fbufs[k], cc)
                    pltpu.async_copy(
                        fbufs[k], acc_sh.at[dst_v.at[cc]], ssems[k], add=True)

                    @pl.when(cc + nbuf < NCH)
                    def _(k=k, cc=cc):
                        pltpu.async_copy(
                            h_hbm.at[src_v.at[cc + nbuf]], gbufs[k], gsems[k])

            for k in range(nbuf):
                pltpu.make_async_copy(
                    fbufs[k], acc_sh.at[dst_v.at[NCH - nbuf + k]],
                    ssems[k]).wait()
        else:
            pltpu.async_copy(h_hbm.at[src_v.at[0]], gbufs[0], gsems[0])
            pltpu.async_copy(h_hbm.at[src_v.at[1]], gbufs[1], gsems[1])

            @pl.loop(0, NCH, step=2)
            def _(c):
                for k in range(2):
                    cc = c + k
                    pltpu.make_async_copy(
                        h_hbm.at[src_v.at[cc]], gbufs[k], gsems[k]).wait()
                    scale(gbufs[k], fbufs[0], cc)
                    pltpu.sync_copy(fbufs[0], acc_sh.at[dst_v.at[cc]],
                                    add=True)

                    @pl.when(cc + 2 < NCH)
                    def _(k=k, cc=cc):
                        pltpu.async_copy(
                            h_hbm.at[src_v.at[cc + 2]], gbufs[k], gsems[k])

        plsc.subcore_barrier()
        for k in range(RPS // CH):
            sl = pl.ds(sid * RPS + k * CH, CH)
            pltpu.sync_copy(acc_sh.at[sl], out_hbm.at[cid].at[sl])

    return conv


_sc_conv64 = _make_conv(64, packed=True)

_C128_SCRATCH = [
    pltpu.VMEM((NCH, CH), jnp.int32),
    pltpu.VMEM((2, CH), jnp.int32),
    pltpu.VMEM((2, CH), jnp.int32),
    pltpu.VMEM((CH, F // 2), jnp.int32),
    pltpu.VMEM((CH, F // 2), jnp.int32),
    pltpu.VMEM((CH, F), jnp.float32),
    pltpu.VMEM_SHARED((NP, F), jnp.float32),
    pltpu.SemaphoreType.DMA,
    pltpu.SemaphoreType.DMA,
    pltpu.SemaphoreType.DMA,
    pltpu.SemaphoreType.DMA,
]


@functools.partial(
    pl.kernel,
    out_type=jax.ShapeDtypeStruct((2, NP, F), jnp.float32),
    mesh=_MESH,
    compiler_params=_SC_PARAMS,
    scratch_types=_C128_SCRATCH,
)
def _sc_conv128(h_hbm, src_hbm, edw_hbm, out_hbm, src_v, dw0_v, dw1_v,
                g0_v, g1_v, f_v, acc_sh, gs0, gs1, es0, es1):
    dwbufs = (dw0_v, dw1_v)
    gbufs = (g0_v, g1_v)
    gsems = (gs0, gs1)
    esems = (es0, es1)
    cid = lax.axis_index("c")
    sid = lax.axis_index("s")
    wid = cid * NS + sid

    @pl.loop(0, CH)
    def _(r):
        for j in range(F // 16):
            f_v[r, pl.ds(j * 16, 16)] = _zero_vec16()

    for k in range(RPS // CH):
        pltpu.sync_copy(f_v, acc_sh.at[pl.ds(sid * RPS + k * CH, CH)])
    plsc.subcore_barrier()

    pltpu.sync_copy(src_hbm.at[wid], src_v)

    hi = jnp.full((16,), -65536, jnp.int32)

    def scale(rows_i, dw_v):
        @pl.loop(0, CH // 16)
        def _(g):
            wv = plsc.bitcast(dw_v[1, pl.ds(g * 16, 16)], jnp.float32)
            for l in range(16):
                sv = lax.broadcast(wv[l], (16,))
                e = g * 16 + l
                for j in range(F // 32):
                    v = rows_i[e, pl.ds(j * 16, 16)]
                    fe = plsc.bitcast(lax.shift_left(v, 16), jnp.float32)
                    fo = plsc.bitcast(v & hi, jnp.float32)
                    f_v[e, pl.ds((2 * j) * 16, 16)] = fe * sv
                    f_v[e, pl.ds((2 * j + 1) * 16, 16)] = fo * sv

    for b in range(2):
        pltpu.async_copy(h_hbm.at[src_v.at[b]], gbufs[b], gsems[b])
        pltpu.async_copy(edw_hbm.at[wid].at[b], dwbufs[b], esems[b])

    @pl.loop(0, NCH, step=2)
    def _(c):
        for k in range(2):
            cc = c + k
            pltpu.make_async_copy(
                h_hbm.at[src_v.at[cc]], gbufs[k], gsems[k]).wait()
            pltpu.make_async_copy(
                edw_hbm.at[wid].at[cc], dwbufs[k], esems[k]).wait()
            scale(gbufs[k], dwbufs[k])
            pltpu.sync_copy(f_v, acc_sh.at[dwbufs[k].at[0]], add=True)

            @pl.when(cc + 2 < NCH)
            def _(k=k, cc=cc):
                pltpu.async_copy(
                    h_hbm.at[src_v.at[cc + 2]], gbufs[k], gsems[k])
                pltpu.async_copy(
                    edw_hbm.at[wid].at[cc + 2], dwbufs[k], esems[k])

    plsc.subcore_barrier()
    for k in range(RPS // CH):
        sl = pl.ds(sid * RPS + k * CH, CH)
        pltpu.sync_copy(acc_sh.at[sl], out_hbm.at[cid].at[sl])


@functools.partial(
    pl.kernel,
    out_type=(
        jax.ShapeDtypeStruct((NW, NCH, CH), jnp.float32),
        jax.ShapeDtypeStruct((NW, NP), jnp.float32),
    ),
    mesh=_MESH,
    compiler_params=_SC_PARAMS,
    scratch_types=[
        pltpu.VMEM((NCH, CH), jnp.int32),
        pltpu.VMEM((NCH, CH), jnp.int32),
        pltpu.VMEM((NCH, CH), jnp.float32),
        pltpu.VMEM((CH, F // 2), jnp.int32),
        pltpu.VMEM((CH, F // 2), jnp.int32),
        pltpu.VMEM((CH, F // 2), jnp.int32),
        pltpu.VMEM((CH, F // 2), jnp.int32),
        pltpu.VMEM((NP,), jnp.float32),
        pltpu.SemaphoreType.DMA,
        pltpu.SemaphoreType.DMA,
        pltpu.SemaphoreType.DMA,
        pltpu.SemaphoreType.DMA,
    ],
)
def _sc_decode(z_hbm, src_hbm, dst_hbm, probs_hbm, degp_hbm,
               src_v, dst_v, probs_v, zs0_v, zd0_v, zs1_v, zd1_v,
               acc_v, ss0, sd0, ss1, sd1):
    cid = lax.axis_index("c")
    sid = lax.axis_index("s")
    wid = cid * NS + sid

    @pl.loop(0, NP // 16)
    def _(i):
        acc_v[pl.ds(i * 16, 16)] = _zero_vec16()

    pltpu.sync_copy(src_hbm.at[wid], src_v)
    pltpu.sync_copy(dst_hbm.at[wid], dst_v)

    lane = lax.iota(jnp.int32, 16)
    ebase = wid * EPW
    perms = {m: lane ^ m for m in (1, 2, 4, 8)}
    masks = {m: (lane & m) == 0 for m in (1, 2, 4, 8)}
    _dn = lax.GatherDimensionNumbers(
        offset_dims=(), collapsed_slice_dims=(0,), start_index_map=(0,))

    def _perm(v, idx):
        return lax.gather(v, idx[:, None], _dn, (1,),
                          mode=lax.GatherScatterMode.PROMISE_IN_BOUNDS)

    def dots_chunk(zs_v, zd_v, c):
        @pl.loop(0, CH // 16)
        def _(g):
            ps = []
            hi = jnp.full((16,), -65536, jnp.int32)
            for l in range(16):
                e = g * 16 + l
                prod = None
                for j in range(F // 32):
                    sl = pl.ds(j * 16, 16)
                    vs = zs_v[e, sl]
                    vd = zd_v[e, sl]
                    se = plsc.bitcast(lax.shift_left(vs, 16), jnp.float32)
                    so = plsc.bitcast(vs & hi, jnp.float32)
                    de = plsc.bitcast(lax.shift_left(vd, 16), jnp.float32)
                    do = plsc.bitcast(vd & hi, jnp.float32)
                    t = se * de + so * do
                    prod = t if prod is None else prod + t
                ps.append(prod)
            for m in (1, 2, 4, 8):
                nxt = []
                for i in range(0, len(ps), 2):
                    a2 = ps[i] + _perm(ps[i], perms[m])
                    b2 = ps[i + 1] + _perm(ps[i + 1], perms[m])
                    nxt.append(jnp.where(masks[m], a2, b2))
                ps = nxt
            dots = ps[0]
            sl = pl.ds(g * 16, 16)
            eid = lane + (ebase + c * CH + g * 16)
            p = jnp.where(eid < E, 1.0 / (1.0 + jnp.exp(-dots)), 0.0)
            probs_v[c, sl] = p
            plsc.addupdate_scatter(acc_v, [dst_v[c, sl]], p)

    pltpu.async_copy(z_hbm.at[src_v.at[0]], zs0_v, ss0)
    pltpu.async_copy(z_hbm.at[dst_v.at[0]], zd0_v, sd0)
    pltpu.async_copy(z_hbm.at[src_v.at[1]], zs1_v, ss1)
    pltpu.async_copy(z_hbm.at[dst_v.at[1]], zd1_v, sd1)

    @pl.loop(0, NCH, step=2)
    def _(c):
        pltpu.make_async_copy(z_hbm.at[src_v.at[c]], zs0_v, ss0).wait()
        pltpu.make_async_copy(z_hbm.at[dst_v.at[c]], zd0_v, sd0).wait()
        dots_chunk(zs0_v, zd0_v, c)

        @pl.when(c + 2 < NCH)
        def _():
            pltpu.async_copy(z_hbm.at[src_v.at[c + 2]], zs0_v, ss0)
            pltpu.async_copy(z_hbm.at[dst_v.at[c + 2]], zd0_v, sd0)

        pltpu.make_async_copy(z_hbm.at[src_v.at[c + 1]], zs1_v, ss1).wait()
        pltpu.make_async_copy(z_hbm.at[dst_v.at[c + 1]], zd1_v, sd1).wait()
        dots_chunk(zs1_v, zd1_v, c + 1)

        @pl.when(c + 3 < NCH)
        def _():
            pltpu.async_copy(z_hbm.at[src_v.at[c + 3]], zs1_v, ss1)
            pltpu.async_copy(z_hbm.at[dst_v.at[c + 3]], zd1_v, sd1)

    pltpu.sync_copy(probs_v, probs_hbm.at[wid])
    pltpu.sync_copy(acc_v, degp_hbm.at[wid])


_BT = 512


def _mm2(x, Wn, Wa):
    M, K = x.shape
    _, Nn = Wn.shape

    def body(x_ref, wn_ref, wa_ref, on_ref, oa_ref):
        xb = x_ref[...]
        on_ref[...] = jnp.dot(xb, wn_ref[...],
                              preferred_element_type=jnp.float32)
        oa_ref[...] = jnp.dot(xb, wa_ref[...],
                              preferred_element_type=jnp.float32)

    return pl.pallas_call(
        body,
        grid=(M // _BT,),
        in_specs=[pl.BlockSpec((_BT, K), lambda i: (i, 0)),
                  pl.BlockSpec((K, Nn), lambda i: (0, 0)),
                  pl.BlockSpec((K, Nn), lambda i: (0, 0))],
        out_specs=[pl.BlockSpec((_BT, Nn), lambda i: (i, 0)),
                   pl.BlockSpec((_BT, Nn), lambda i: (i, 0))],
        out_shape=[jax.ShapeDtypeStruct((M, Nn), jnp.float32),
                   jax.ShapeDtypeStruct((M, Nn), jnp.float32)],
    )(x, Wn, Wa)


def _norms(degp, h1n, h1a):
    def body(dp_ref, h1n_ref, h1a_ref, h1p_ref, h1ai_ref, dis_ref, invd_ref):
        deg = jnp.sum(dp_ref[...], axis=0, keepdims=True) + 1.0
        dis = lax.rsqrt(deg)
        invd = 1.0 / deg
        dis_ref[...] = dis.T
        invd_ref[...] = invd.T
        h1p_ref[...] = (h1n_ref[...] * dis.T).astype(jnp.bfloat16)
        h1ai_ref[...] = h1a_ref[...] * invd.T

    return pl.pallas_call(
        body,
        grid=(NP // _BT,),
        in_specs=[pl.BlockSpec((NW, _BT), lambda i: (0, i)),
                  pl.BlockSpec((_BT, H), lambda i: (i, 0)),
                  pl.BlockSpec((_BT, H), lambda i: (i, 0))],
        out_specs=[pl.BlockSpec((_BT, H), lambda i: (i, 0)),
                   pl.BlockSpec((_BT, H), lambda i: (i, 0)),
                   pl.BlockSpec((_BT, 1), lambda i: (i, 0)),
                   pl.BlockSpec((_BT, 1), lambda i: (i, 0))],
        out_shape=[jax.ShapeDtypeStruct((NP, H), jnp.bfloat16),
                   jax.ShapeDtypeStruct((NP, H), jnp.float32),
                   jax.ShapeDtypeStruct((NP, 1), jnp.float32),
                   jax.ShapeDtypeStruct((NP, 1), jnp.float32)],
    )(degp, h1n, h1a)


def _post1(acc, h1ai, dis, invd, b1t, W2s, W2a):
    def body(acc_ref, h1ai_ref, dis_ref, invd_ref, b1_ref, w2s_ref,
             w2a_ref, h2p_ref, h2i_ref):
        s = acc_ref[0] + acc_ref[1]
        h = jnp.maximum(
            dis_ref[...] * s + h1ai_ref[...] + b1_ref[...], 0.0)
        h2n = jnp.dot(h, w2s_ref[...], preferred_element_type=jnp.float32)
        h2a = jnp.dot(h, w2a_ref[...], preferred_element_type=jnp.float32)
        h2p_ref[...] = (h2n * dis_ref[...]).astype(jnp.bfloat16)
        h2i_ref[...] = h2a * invd_ref[...]

    return pl.pallas_call(
        body,
        grid=(NP // _BT,),
        in_specs=[pl.BlockSpec((2, _BT, H), lambda i: (0, i, 0)),
                  pl.BlockSpec((_BT, H), lambda i: (i, 0)),
                  pl.BlockSpec((_BT, 1), lambda i: (i, 0)),
                  pl.BlockSpec((_BT, 1), lambda i: (i, 0)),
                  pl.BlockSpec((1, H), lambda i: (0, 0)),
                  pl.BlockSpec((H, F), lambda i: (0, 0)),
                  pl.BlockSpec((H, F), lambda i: (0, 0))],
        out_specs=[pl.BlockSpec((_BT, F), lambda i: (i, 0)),
                   pl.BlockSpec((_BT, F), lambda i: (i, 0))],
        out_shape=[jax.ShapeDtypeStruct((NP, F), jnp.bfloat16),
                   jax.ShapeDtypeStruct((NP, F), jnp.float32)],
    )(acc, h1ai, dis, invd, b1t, W2s, W2a)


def _post2(acc, h2i, dis, b2, x, Wd, Wda):
    def body(acc_ref, h2i_ref, dis_ref, b2_ref, x_ref, wd_ref, wda_ref,
             z_ref, yn_ref, ya_ref):
        s = acc_ref[0] + acc_ref[1]
        logits = dis_ref[...] * s + h2i_ref[...] + b2_ref[...]
        z = 1.0 / (1.0 + jnp.exp(-logits))
        z_ref[...] = z.astype(jnp.bfloat16)
        zres = z + x_ref[...]
        yn_ref[...] = jnp.dot(zres, wd_ref[...],
                              preferred_element_type=jnp.float32)
        ya_ref[...] = jnp.dot(zres, wda_ref[...],
                              preferred_element_type=jnp.float32)

    return pl.pallas_call(
        body,
        grid=(NP // _BT,),
        in_specs=[pl.BlockSpec((2, _BT, F), lambda i: (0, i, 0)),
                  pl.BlockSpec((_BT, F), lambda i: (i, 0)),
                  pl.BlockSpec((_BT, 1), lambda i: (i, 0)),
                  pl.BlockSpec((1, F), lambda i: (0, 0)),
                  pl.BlockSpec((_BT, F), lambda i: (i, 0)),
                  pl.BlockSpec((F, H), lambda i: (0, 0)),
                  pl.BlockSpec((F, H), lambda i: (0, 0))],
        out_specs=[pl.BlockSpec((_BT, F), lambda i: (i, 0)),
                   pl.BlockSpec((_BT, H), lambda i: (i, 0)),
                   pl.BlockSpec((_BT, H), lambda i: (i, 0))],
        out_shape=[jax.ShapeDtypeStruct((NP, F), jnp.bfloat16),
                   jax.ShapeDtypeStruct((NP, H), jnp.float32),
                   jax.ShapeDtypeStruct((NP, H), jnp.float32)],
    )(acc, h2i, dis, b2, x, Wd, Wda)


def _norms2(degp, yn, ya):
    def body(dp_ref, yn_ref, ya_ref, yp_ref, yai_ref, dis_ref):
        deg = jnp.sum(dp_ref[...], axis=0, keepdims=True) + 1.0
        dis = lax.rsqrt(deg)
        invd = 1.0 / deg
        yp_ref[...] = (yn_ref[...] * dis.T).astype(jnp.bfloat16)
        yai_ref[...] = ya_ref[...] * invd.T
        dis_ref[...] = dis.T

    return pl.pallas_call(
        body,
        grid=(NP // _BT,),
        in_specs=[pl.BlockSpec((NW, _BT), lambda i: (0, i)),
                  pl.BlockSpec((_BT, H), lambda i: (i, 0)),
                  pl.BlockSpec((_BT, H), lambda i: (i, 0))],
        out_specs=[pl.BlockSpec((_BT, H), lambda i: (i, 0)),
                   pl.BlockSpec((_BT, H), lambda i: (i, 0)),
                   pl.BlockSpec((_BT, 1), lambda i: (i, 0))],
        out_shape=[jax.ShapeDtypeStruct((NP, H), jnp.bfloat16),
                   jax.ShapeDtypeStruct((NP, H), jnp.float32),
                   jax.ShapeDtypeStruct((NP, 1), jnp.float32)],
    )(degp, yn, ya)


def _head(acc, yai, dis2, bdt, Wfct, bfc):
    nsteps = NP // _BT

    def body(acc_ref, yai_ref, dis_ref, bd_ref, wfc_ref, bfc_ref, o_ref):
        i = pl.program_id(0)

        @pl.when(i == 0)
        def _():
            o_ref[...] = jnp.zeros((1, 1), jnp.float32)

        s = acc_ref[0] + acc_ref[1]
        hd = jnp.maximum(
            dis_ref[...] * s + yai_ref[...] + bd_ref[...], 0.0)
        o_ref[...] += jnp.sum(hd * wfc_ref[...]).reshape(1, 1)

        @pl.when(i == nsteps - 1)
        def _():
            t = o_ref[...] + bfc_ref[...]
            o_ref[...] = 1.0 / (1.0 + jnp.exp(-t))

    return pl.pallas_call(
        body,
        grid=(nsteps,),
        in_specs=[pl.BlockSpec((2, _BT, H), lambda i: (0, i, 0)),
                  pl.BlockSpec((_BT, H), lambda i: (i, 0)),
                  pl.BlockSpec((_BT, 1), lambda i: (i, 0)),
                  pl.BlockSpec((1, H), lambda i: (0, 0)),
                  pl.BlockSpec((_BT, H), lambda i: (i, 0)),
                  pl.BlockSpec((1, 1), lambda i: (0, 0))],
        out_specs=pl.BlockSpec((1, 1), lambda i: (0, 0)),
        out_shape=jax.ShapeDtypeStruct((1, 1), jnp.float32),
    )(acc, yai, dis2, bdt, Wfct, bfc)


def kernel(features, edge_index, edge_attr, W1, b1, W2, b2, Wd, bd, Wfc, bfc):
    pad = EP - E
    src = jnp.concatenate(
        [edge_index[0].astype(jnp.int32), jnp.zeros((pad,), jnp.int32)]
    ).reshape(NW, NCH, CH)
    dst = jnp.concatenate(
        [edge_index[1].astype(jnp.int32), jnp.zeros((pad,), jnp.int32)]
    ).reshape(NW, NCH, CH)
    w = jnp.concatenate(
        [edge_attr.astype(jnp.float32), jnp.zeros((pad,), jnp.float32)]
    ).reshape(NW, NCH, CH)

    xP = jnp.pad(features, ((0, NP - N), (0, 0)))
    W1a = W1[:, _U64]
    b1t = b1[_U64].reshape(1, H)
    W2s = W2[_U64, :]
    W2a = W2s[:, _U128]
    b2t = b2[_U128].reshape(1, F)
    xPt = xP[:, _U128]
    Wd1 = Wd[_U128, :]
    Wda = Wd1[:, _U64]
    bdt = bd[_U64].reshape(1, H)
    WfcP = jnp.pad(Wfc.reshape(N, H), ((0, NP - N), (0, 0)))
    Wfct = WfcP[:, _U64]
    bfcr = bfc.reshape(1, 1)

    degp = _sc_deg(dst, w)
    h1n, h1a = _mm2(xP, W1, W1a)
    h1p_bf, h1ai, dis, invd = _norms(degp, h1n, h1a)
    acc1 = _sc_conv64(_pack_bf16(h1p_bf), src, dst, w)
    h2p_bf, h2i = _post1(acc1, h1ai, dis, invd, b1t, W2s, W2a)
    edw = jnp.stack([dst, lax.bitcast_convert_type(w, jnp.int32)], axis=2)
    acc2 = _sc_conv128(_pack_bf16(h2p_bf), src, edw)
    z_bf, yn, ya = _post2(acc2, h2i, dis, b2t, xPt, Wd1, Wda)
    probs, degp2 = _sc_decode(_pack_bf16(z_bf), src, dst)
    yp_bf, yai, dis2 = _norms2(degp2, yn, ya)
    acc3 = _sc_conv64(_pack_bf16(yp_bf), src, dst, probs)
    out = _head(acc3, yai, dis2, bdt, Wfct, bfcr)
    return out[0, 0]


# speedup vs baseline: 1.7433x; 1.0027x over previous
import dataclasses
import functools

import numpy as np

import jax
import jax.numpy as jnp
from jax import lax
from jax.experimental import pallas as pl
from jax.experimental.pallas import tpu as pltpu
from jax.experimental.pallas import tpu_sc as plsc

N = 10000
NP = 10240
F = 128
H = 64
E = 320000
NS = 16
NW = 32
EPW = 10240
CH = 128
NCH = EPW // CH
EP = NW * EPW
RPS = NP // NS

_MESH = plsc.VectorSubcoreMesh(
    core_axis_name="c", subcore_axis_name="s", num_cores=2, num_subcores=16
)

_SC_PARAMS = pltpu.CompilerParams()
if "needs_layout_passes" in pltpu.CompilerParams.__dataclass_fields__:
    _SC_PARAMS = dataclasses.replace(_SC_PARAMS, needs_layout_passes=False)
if "use_tc_tiling_on_sc" in pltpu.CompilerParams.__dataclass_fields__:
    _SC_PARAMS = dataclasses.replace(_SC_PARAMS, use_tc_tiling_on_sc=False)


def _unpack_perm(dim):
    out = []
    for b in range(dim // 32):
        out.extend(range(32 * b, 32 * b + 32, 2))
        out.extend(range(32 * b + 1, 32 * b + 32, 2))
    return np.array(out)


_U64 = _unpack_perm(64)
_U128 = _unpack_perm(128)


def _zero_vec16():
    return jnp.zeros((16,), jnp.float32)


def _pack_bf16(x_bf):
    m, d = x_bf.shape
    return lax.bitcast_convert_type(x_bf.reshape(m, d // 2, 2), jnp.int32)


@functools.partial(
    pl.kernel,
    out_type=jax.ShapeDtypeStruct((NW, NP), jnp.float32),
    mesh=_MESH,
    compiler_params=_SC_PARAMS,
    scratch_types=[
        pltpu.VMEM((NCH, CH), jnp.int32),
        pltpu.VMEM((NCH, CH), jnp.float32),
        pltpu.VMEM((NP,), jnp.float32),
    ],
)
def _sc_deg(dst_hbm, w_hbm, out_hbm, dst_v, w_v, acc_v):
    cid = lax.axis_index("c")
    sid = lax.axis_index("s")
    wid = cid * NS + sid

    @pl.loop(0, NP // 16)
    def _(i):
        acc_v[pl.ds(i * 16, 16)] = _zero_vec16()

    pltpu.sync_copy(dst_hbm.at[wid], dst_v)
    pltpu.sync_copy(w_hbm.at[wid], w_v)

    @pl.loop(0, NCH)
    def _(c):
        for j in range(CH // 16):
            idx = dst_v[c, pl.ds(j * 16, 16)]
            val = w_v[c, pl.ds(j * 16, 16)]
            plsc.addupdate_scatter(acc_v, [idx], val)

    pltpu.sync_copy(acc_v, out_hbm.at[wid])


def _make_conv(dim, packed):
    assert packed
    nbuf = 4 if dim <= 64 else 2
    nfb = nbuf if dim <= 64 else 1
    gdim = dim // 2
    scratch = [
        pltpu.VMEM((NCH, CH), jnp.int32),
        pltpu.VMEM((NCH, CH), jnp.int32),
        pltpu.VMEM((NCH, CH), jnp.float32),
    ]
    scratch += [pltpu.VMEM((CH, gdim), jnp.int32) for _ in range(nbuf)]
    scratch += [pltpu.VMEM((CH, dim), jnp.float32) for _ in range(nfb)]
    scratch += [pltpu.VMEM_SHARED((NP, dim), jnp.float32)]
    scratch += [pltpu.SemaphoreType.DMA for _ in range(2 * nbuf)]

    @functools.partial(
        pl.kernel,
        out_type=jax.ShapeDtypeStruct((2, NP, dim), jnp.float32),
        mesh=_MESH,
        compiler_params=_SC_PARAMS,
        scratch_types=scratch,
    )
    def conv(h_hbm, src_hbm, dst_hbm, w_hbm, out_hbm, src_v, dst_v, w_v,
             *rest):
        gbufs = rest[:nbuf]
        rest = rest[nbuf:]
        fbufs = rest[:nfb]
        rest = rest[nfb:]
        acc_sh = rest[0]
        gsems = rest[1:nbuf + 1]
        ssems = rest[nbuf + 1:]
        cid = lax.axis_index("c")
        sid = lax.axis_index("s")
        wid = cid * NS + sid

        @pl.loop(0, CH)
        def _(r):
            for j in range(dim // 16):
                fbufs[0][r, pl.ds(j * 16, 16)] = _zero_vec16()

        for k in range(RPS // CH):
            pltpu.sync_copy(fbufs[0], acc_sh.at[pl.ds(sid * RPS + k * CH, CH)])
        plsc.subcore_barrier()

        pltpu.sync_copy(src_hbm.at[wid], src_v)
        pltpu.sync_copy(dst_hbm.at[wid], dst_v)
        pltpu.sync_copy(w_hbm.at[wid], w_v)

        hi = jnp.full((16,), -65536, jnp.int32)

        def scale(rows_i, rows_f, c):
            @pl.loop(0, CH // 16)
            def _(g):
                wv = w_v[c, pl.ds(g * 16, 16)]
                for l in range(16):
                    sv = lax.broadcast(wv[l], (16,))
                    e = g * 16 + l
                    for j in range(dim // 32):
                        v = rows_i[e, pl.ds(j * 16, 16)]
                        fe = plsc.bitcast(lax.shift_left(v, 16),
                                          jnp.float32)
                        fo = plsc.bitcast(v & hi, jnp.float32)
                        rows_f[e, pl.ds((2 * j) * 16, 16)] = fe * sv
                        rows_f[e, pl.ds((2 * j + 1) * 16, 16)] = fo * sv

        if nfb == nbuf:
            for b in range(nbuf):
                pltpu.async_copy(h_hbm.at[src_v.at[b]], gbufs[b], gsems[b])

            @pl.loop(0, NCH, step=nbuf)
            def _(c):
                for k in range(nbuf):
                    cc = c + k
                    pltpu.make_async_copy(
                        h_hbm.at[src_v.at[cc]], gbufs[k], gsems[k]).wait()

                    @pl.when(cc >= nbuf)
                    def _(k=k, cc=cc):
                        pltpu.make_async_copy(
                            fbufs[k], acc_sh.at[dst_v.at[cc - nbuf]],
                            ssems[k]).wait()

                    scale(gbufs[k], fbufs[k], cc)
                    pltpu.async_copy(
                        fbufs[k], acc_sh.at[dst_v.at[cc]], ssems[k], add=True)

                    @pl.when(cc + nbuf < NCH)
                    def _(k=k, cc=cc):
                        pltpu.async_copy(
                            h_hbm.at[src_v.at[cc + nbuf]], gbufs[k], gsems[k])

            for k in range(nbuf):
                pltpu.make_async_copy(
                    fbufs[k], acc_sh.at[dst_v.at[NCH - nbuf + k]],
                    ssems[k]).wait()
        else:
            pltpu.async_copy(h_hbm.at[src_v.at[0]], gbufs[0], gsems[0])
            pltpu.async_copy(h_hbm.at[src_v.at[1]], gbufs[1], gsems[1])

            @pl.loop(0, NCH, step=2)
            def _(c):
                for k in range(2):
                    cc = c + k
                    pltpu.make_async_copy(
                        h_hbm.at[src_v.at[cc]], gbufs[k], gsems[k]).wait()
                    scale(gbufs[k], fbufs[0], cc)
                    pltpu.sync_copy(fbufs[0], acc_sh.at[dst_v.at[cc]],
                                    add=True)

                    @pl.when(cc + 2 < NCH)
                    def _(k=k, cc=cc):
                        pltpu.async_copy(
                            h_hbm.at[src_v.at[cc + 2]], gbufs[k], gsems[k])

        plsc.subcore_barrier()
        for k in range(RPS // CH):
            sl = pl.ds(sid * RPS + k * CH, CH)
            pltpu.sync_copy(acc_sh.at[sl], out_hbm.at[cid].at[sl])

    return conv


_sc_conv64 = _make_conv(64, packed=True)

_C128_SCRATCH = [
    pltpu.VMEM((NCH, CH), jnp.int32),
    pltpu.VMEM((2, CH), jnp.int32),
    pltpu.VMEM((2, CH), jnp.int32),
    pltpu.VMEM((CH, F // 2), jnp.int32),
    pltpu.VMEM((CH, F // 2), jnp.int32),
    pltpu.VMEM((CH, F), jnp.float32),
    pltpu.VMEM_SHARED((NP, F), jnp.float32),
    pltpu.SemaphoreType.DMA,
    pltpu.SemaphoreType.DMA,
    pltpu.SemaphoreType.DMA,
    pltpu.SemaphoreType.DMA,
]


@functools.partial(
    pl.kernel,
    out_type=jax.ShapeDtypeStruct((2, NP, F), jnp.float32),
    mesh=_MESH,
    compiler_params=_SC_PARAMS,
    scratch_types=_C128_SCRATCH,
)
def _sc_conv128(h_hbm, src_hbm, edw_hbm, out_hbm, src_v, dw0_v, dw1_v,
                g0_v, g1_v, f_v, acc_sh, gs0, gs1, es0, es1):
    dwbufs = (dw0_v, dw1_v)
    gbufs = (g0_v, g1_v)
    gsems = (gs0, gs1)
    esems = (es0, es1)
    cid = lax.axis_index("c")
    sid = lax.axis_index("s")
    wid = cid * NS + sid

    @pl.loop(0, CH)
    def _(r):
        for j in range(F // 16):
            f_v[r, pl.ds(j * 16, 16)] = _zero_vec16()

    for k in range(RPS // CH):
        pltpu.sync_copy(f_v, acc_sh.at[pl.ds(sid * RPS + k * CH, CH)])
    plsc.subcore_barrier()

    pltpu.sync_copy(src_hbm.at[wid], src_v)

    hi = jnp.full((16,), -65536, jnp.int32)

    def scale(rows_i, dw_v):
        @pl.loop(0, CH // 16)
        def _(g):
            wv = plsc.bitcast(dw_v[1, pl.ds(g * 16, 16)], jnp.float32)
            for l in range(16):
                sv = lax.broadcast(wv[l], (16,))
                e = g * 16 + l
                for j in range(F // 32):
                    v = rows_i[e, pl.ds(j * 16, 16)]
                    fe = plsc.bitcast(lax.shift_left(v, 16), jnp.float32)
                    fo = plsc.bitcast(v & hi, jnp.float32)
                    f_v[e, pl.ds((2 * j) * 16, 16)] = fe * sv
                    f_v[e, pl.ds((2 * j + 1) * 16, 16)] = fo * sv

    for b in range(2):
        pltpu.async_copy(h_hbm.at[src_v.at[b]], gbufs[b], gsems[b])
        pltpu.async_copy(edw_hbm.at[wid].at[b], dwbufs[b], esems[b])

    @pl.loop(0, NCH, step=2)
    def _(c):
        for k in range(2):
            cc = c + k
            pltpu.make_async_copy(
                h_hbm.at[src_v.at[cc]], gbufs[k], gsems[k]).wait()
            pltpu.make_async_copy(
                edw_hbm.at[wid].at[cc], dwbufs[k], esems[k]).wait()
            scale(gbufs[k], dwbufs[k])
            pltpu.sync_copy(f_v, acc_sh.at[dwbufs[k].at[0]], add=True)

            @pl.when(cc + 2 < NCH)
            def _(k=k, cc=cc):
                pltpu.async_copy(
                    h_hbm.at[src_v.at[cc + 2]], gbufs[k], gsems[k])
                pltpu.async_copy(
                    edw_hbm.at[wid].at[cc + 2], dwbufs[k], esems[k])

    plsc.subcore_barrier()
    for k in range(RPS // CH):
        sl = pl.ds(sid * RPS + k * CH, CH)
        pltpu.sync_copy(acc_sh.at[sl], out_hbm.at[cid].at[sl])


@functools.partial(
    pl.kernel,
    out_type=(
        jax.ShapeDtypeStruct((NW, NCH, CH), jnp.float32),
        jax.ShapeDtypeStruct((NW, NP), jnp.float32),
    ),
    mesh=_MESH,
    compiler_params=_SC_PARAMS,
    scratch_types=[
        pltpu.VMEM((NCH, CH), jnp.int32),
        pltpu.VMEM((NCH, CH), jnp.int32),
        pltpu.VMEM((NCH, CH), jnp.float32),
    ] + [pltpu.VMEM((CH, F // 2), jnp.int32) for _ in range(8)] + [
        pltpu.VMEM((NP,), jnp.float32),
    ] + [pltpu.SemaphoreType.DMA for _ in range(8)],
)
def _sc_decode(z_hbm, src_hbm, dst_hbm, probs_hbm, degp_hbm,
               src_v, dst_v, probs_v, *rest):
    zsb = rest[0:4]
    zdb = rest[4:8]
    acc_v = rest[8]
    ssems = rest[9:13]
    dsems = rest[13:17]
    cid = lax.axis_index("c")
    sid = lax.axis_index("s")
    wid = cid * NS + sid

    @pl.loop(0, NP // 16)
    def _(i):
        acc_v[pl.ds(i * 16, 16)] = _zero_vec16()

    pltpu.sync_copy(src_hbm.at[wid], src_v)
    pltpu.sync_copy(dst_hbm.at[wid], dst_v)

    lane = lax.iota(jnp.int32, 16)
    ebase = wid * EPW
    perms = {m: lane ^ m for m in (1, 2, 4, 8)}
    masks = {m: (lane & m) == 0 for m in (1, 2, 4, 8)}
    _dn = lax.GatherDimensionNumbers(
        offset_dims=(), collapsed_slice_dims=(0,), start_index_map=(0,))

    def _perm(v, idx):
        return lax.gather(v, idx[:, None], _dn, (1,),
                          mode=lax.GatherScatterMode.PROMISE_IN_BOUNDS)

    def dots_chunk(zs_v, zd_v, c):
        @pl.loop(0, CH // 16)
        def _(g):
            ps = []
            hi = jnp.full((16,), -65536, jnp.int32)
            for l in range(16):
                e = g * 16 + l
                prod = None
                for j in range(F // 32):
                    sl = pl.ds(j * 16, 16)
                    vs = zs_v[e, sl]
                    vd = zd_v[e, sl]
                    se = plsc.bitcast(lax.shift_left(vs, 16), jnp.float32)
                    so = plsc.bitcast(vs & hi, jnp.float32)
                    de = plsc.bitcast(lax.shift_left(vd, 16), jnp.float32)
                    do = plsc.bitcast(vd & hi, jnp.float32)
                    t = se * de + so * do
                    prod = t if prod is None else prod + t
                ps.append(prod)
            for m in (1, 2, 4, 8):
                nxt = []
                for i in range(0, len(ps), 2):
                    a2 = ps[i] + _perm(ps[i], perms[m])
                    b2 = ps[i + 1] + _perm(ps[i + 1], perms[m])
                    nxt.append(jnp.where(masks[m], a2, b2))
                ps = nxt
            dots = ps[0]
            sl = pl.ds(g * 16, 16)
            eid = lane + (ebase + c * CH + g * 16)
            p = jnp.where(eid < E, 1.0 / (1.0 + jnp.exp(-dots)), 0.0)
            probs_v[c, sl] = p
            plsc.addupdate_scatter(acc_v, [dst_v[c, sl]], p)

    for b in range(4):
        pltpu.async_copy(z_hbm.at[src_v.at[b]], zsb[b], ssems[b])
        pltpu.async_copy(z_hbm.at[dst_v.at[b]], zdb[b], dsems[b])

    @pl.loop(0, NCH, step=4)
    def _(c):
        for k in range(4):
            cc = c + k
            pltpu.make_async_copy(z_hbm.at[src_v.at[cc]], zsb[k],
                                  ssems[k]).wait()
            pltpu.make_async_copy(z_hbm.at[dst_v.at[cc]], zdb[k],
                                  dsems[k]).wait()
            dots_chunk(zsb[k], zdb[k], cc)

            @pl.when(cc + 4 < NCH)
            def _(k=k, cc=cc):
                pltpu.async_copy(z_hbm.at[src_v.at[cc + 4]], zsb[k], ssems[k])
                pltpu.async_copy(z_hbm.at[dst_v.at[cc + 4]], zdb[k], dsems[k])

    pltpu.sync_copy(probs_v, probs_hbm.at[wid])
    pltpu.sync_copy(acc_v, degp_hbm.at[wid])


_BT = 512


def _mm2(x, Wn, Wa):
    M, K = x.shape
    _, Nn = Wn.shape

    def body(x_ref, wn_ref, wa_ref, on_ref, oa_ref):
        xb = x_ref[...]
        on_ref[...] = jnp.dot(xb, wn_ref[...],
                              preferred_element_type=jnp.float32)
        oa_ref[...] = jnp.dot(xb, wa_ref[...],
                              preferred_element_type=jnp.float32)

    return pl.pallas_call(
        body,
        grid=(M // _BT,),
        in_specs=[pl.BlockSpec((_BT, K), lambda i: (i, 0)),
                  pl.BlockSpec((K, Nn), lambda i: (0, 0)),
                  pl.BlockSpec((K, Nn), lambda i: (0, 0))],
        out_specs=[pl.BlockSpec((_BT, Nn), lambda i: (i, 0)),
                   pl.BlockSpec((_BT, Nn), lambda i: (i, 0))],
        out_shape=[jax.ShapeDtypeStruct((M, Nn), jnp.float32),
                   jax.ShapeDtypeStruct((M, Nn), jnp.float32)],
    )(x, Wn, Wa)


def _norms(degp, h1n, h1a):
    def body(dp_ref, h1n_ref, h1a_ref, h1p_ref, h1ai_ref, dis_ref, invd_ref):
        deg = jnp.sum(dp_ref[...], axis=0, keepdims=True) + 1.0
        dis = lax.rsqrt(deg)
        invd = 1.0 / deg
        dis_ref[...] = dis.T
        invd_ref[...] = invd.T
        h1p_ref[...] = (h1n_ref[...] * dis.T).astype(jnp.bfloat16)
        h1ai_ref[...] = h1a_ref[...] * invd.T

    return pl.pallas_call(
        body,
        grid=(NP // _BT,),
        in_specs=[pl.BlockSpec((NW, _BT), lambda i: (0, i)),
                  pl.BlockSpec((_BT, H), lambda i: (i, 0)),
                  pl.BlockSpec((_BT, H), lambda i: (i, 0))],
        out_specs=[pl.BlockSpec((_BT, H), lambda i: (i, 0)),
                   pl.BlockSpec((_BT, H), lambda i: (i, 0)),
                   pl.BlockSpec((_BT, 1), lambda i: (i, 0)),
                   pl.BlockSpec((_BT, 1), lambda i: (i, 0))],
        out_shape=[jax.ShapeDtypeStruct((NP, H), jnp.bfloat16),
                   jax.ShapeDtypeStruct((NP, H), jnp.float32),
                   jax.ShapeDtypeStruct((NP, 1), jnp.float32),
                   jax.ShapeDtypeStruct((NP, 1), jnp.float32)],
    )(degp, h1n, h1a)


def _post1(acc, h1ai, dis, invd, b1t, W2s, W2a):
    def body(acc_ref, h1ai_ref, dis_ref, invd_ref, b1_ref, w2s_ref,
             w2a_ref, h2p_ref, h2i_ref):
        s = acc_ref[0] + acc_ref[1]
        h = jnp.maximum(
            dis_ref[...] * s + h1ai_ref[...] + b1_ref[...], 0.0)
        h2n = jnp.dot(h, w2s_ref[...], preferred_element_type=jnp.float32)
        h2a = jnp.dot(h, w2a_ref[...], preferred_element_type=jnp.float32)
        h2p_ref[...] = (h2n * dis_ref[...]).astype(jnp.bfloat16)
        h2i_ref[...] = h2a * invd_ref[...]

    return pl.pallas_call(
        body,
        grid=(NP // _BT,),
        in_specs=[pl.BlockSpec((2, _BT, H), lambda i: (0, i, 0)),
                  pl.BlockSpec((_BT, H), lambda i: (i, 0)),
                  pl.BlockSpec((_BT, 1), lambda i: (i, 0)),
                  pl.BlockSpec((_BT, 1), lambda i: (i, 0)),
                  pl.BlockSpec((1, H), lambda i: (0, 0)),
                  pl.BlockSpec((H, F), lambda i: (0, 0)),
                  pl.BlockSpec((H, F), lambda i: (0, 0))],
        out_specs=[pl.BlockSpec((_BT, F), lambda i: (i, 0)),
                   pl.BlockSpec((_BT, F), lambda i: (i, 0))],
        out_shape=[jax.ShapeDtypeStruct((NP, F), jnp.bfloat16),
                   jax.ShapeDtypeStruct((NP, F), jnp.float32)],
    )(acc, h1ai, dis, invd, b1t, W2s, W2a)


def _post2(acc, h2i, dis, b2, x, Wd, Wda):
    def body(acc_ref, h2i_ref, dis_ref, b2_ref, x_ref, wd_ref, wda_ref,
             z_ref, yn_ref, ya_ref):
        s = acc_ref[0] + acc_ref[1]
        logits = dis_ref[...] * s + h2i_ref[...] + b2_ref[...]
        z = 1.0 / (1.0 + jnp.exp(-logits))
        z_ref[...] = z.astype(jnp.bfloat16)
        zres = z + x_ref[...]
        yn_ref[...] = jnp.dot(zres, wd_ref[...],
                              preferred_element_type=jnp.float32)
        ya_ref[...] = jnp.dot(zres, wda_ref[...],
                              preferred_element_type=jnp.float32)

    return pl.pallas_call(
        body,
        grid=(NP // _BT,),
        in_specs=[pl.BlockSpec((2, _BT, F), lambda i: (0, i, 0)),
                  pl.BlockSpec((_BT, F), lambda i: (i, 0)),
                  pl.BlockSpec((_BT, 1), lambda i: (i, 0)),
                  pl.BlockSpec((1, F), lambda i: (0, 0)),
                  pl.BlockSpec((_BT, F), lambda i: (i, 0)),
                  pl.BlockSpec((F, H), lambda i: (0, 0)),
                  pl.BlockSpec((F, H), lambda i: (0, 0))],
        out_specs=[pl.BlockSpec((_BT, F), lambda i: (i, 0)),
                   pl.BlockSpec((_BT, H), lambda i: (i, 0)),
                   pl.BlockSpec((_BT, H), lambda i: (i, 0))],
        out_shape=[jax.ShapeDtypeStruct((NP, F), jnp.bfloat16),
                   jax.ShapeDtypeStruct((NP, H), jnp.float32),
                   jax.ShapeDtypeStruct((NP, H), jnp.float32)],
    )(acc, h2i, dis, b2, x, Wd, Wda)


def _norms2(degp, yn, ya):
    def body(dp_ref, yn_ref, ya_ref, yp_ref, yai_ref, dis_ref):
        deg = jnp.sum(dp_ref[...], axis=0, keepdims=True) + 1.0
        dis = lax.rsqrt(deg)
        invd = 1.0 / deg
        yp_ref[...] = (yn_ref[...] * dis.T).astype(jnp.bfloat16)
        yai_ref[...] = ya_ref[...] * invd.T
        dis_ref[...] = dis.T

    return pl.pallas_call(
        body,
        grid=(NP // _BT,),
        in_specs=[pl.BlockSpec((NW, _BT), lambda i: (0, i)),
                  pl.BlockSpec((_BT, H), lambda i: (i, 0)),
                  pl.BlockSpec((_BT, H), lambda i: (i, 0))],
        out_specs=[pl.BlockSpec((_BT, H), lambda i: (i, 0)),
                   pl.BlockSpec((_BT, H), lambda i: (i, 0)),
                   pl.BlockSpec((_BT, 1), lambda i: (i, 0))],
        out_shape=[jax.ShapeDtypeStruct((NP, H), jnp.bfloat16),
                   jax.ShapeDtypeStruct((NP, H), jnp.float32),
                   jax.ShapeDtypeStruct((NP, 1), jnp.float32)],
    )(degp, yn, ya)


def _head(acc, yai, dis2, bdt, Wfct, bfc):
    nsteps = NP // _BT

    def body(acc_ref, yai_ref, dis_ref, bd_ref, wfc_ref, bfc_ref, o_ref):
        i = pl.program_id(0)

        @pl.when(i == 0)
        def _():
            o_ref[...] = jnp.zeros((1, 1), jnp.float32)

        s = acc_ref[0] + acc_ref[1]
        hd = jnp.maximum(
            dis_ref[...] * s + yai_ref[...] + bd_ref[...], 0.0)
        o_ref[...] += jnp.sum(hd * wfc_ref[...]).reshape(1, 1)

        @pl.when(i == nsteps - 1)
        def _():
            t = o_ref[...] + bfc_ref[...]
            o_ref[...] = 1.0 / (1.0 + jnp.exp(-t))

    return pl.pallas_call(
        body,
        grid=(nsteps,),
        in_specs=[pl.BlockSpec((2, _BT, H), lambda i: (0, i, 0)),
                  pl.BlockSpec((_BT, H), lambda i: (i, 0)),
                  pl.BlockSpec((_BT, 1), lambda i: (i, 0)),
                  pl.BlockSpec((1, H), lambda i: (0, 0)),
                  pl.BlockSpec((_BT, H), lambda i: (i, 0)),
                  pl.BlockSpec((1, 1), lambda i: (0, 0))],
        out_specs=pl.BlockSpec((1, 1), lambda i: (0, 0)),
        out_shape=jax.ShapeDtypeStruct((1, 1), jnp.float32),
    )(acc, yai, dis2, bdt, Wfct, bfc)


def kernel(features, edge_index, edge_attr, W1, b1, W2, b2, Wd, bd, Wfc, bfc):
    pad = EP - E
    src = jnp.concatenate(
        [edge_index[0].astype(jnp.int32), jnp.zeros((pad,), jnp.int32)]
    ).reshape(NW, NCH, CH)
    dst = jnp.concatenate(
        [edge_index[1].astype(jnp.int32), jnp.zeros((pad,), jnp.int32)]
    ).reshape(NW, NCH, CH)
    w = jnp.concatenate(
        [edge_attr.astype(jnp.float32), jnp.zeros((pad,), jnp.float32)]
    ).reshape(NW, NCH, CH)

    xP = jnp.pad(features, ((0, NP - N), (0, 0)))
    W1a = W1[:, _U64]
    b1t = b1[_U64].reshape(1, H)
    W2s = W2[_U64, :]
    W2a = W2s[:, _U128]
    b2t = b2[_U128].reshape(1, F)
    xPt = xP[:, _U128]
    Wd1 = Wd[_U128, :]
    Wda = Wd1[:, _U64]
    bdt = bd[_U64].reshape(1, H)
    WfcP = jnp.pad(Wfc.reshape(N, H), ((0, NP - N), (0, 0)))
    Wfct = WfcP[:, _U64]
    bfcr = bfc.reshape(1, 1)

    degp = _sc_deg(dst, w)
    h1n, h1a = _mm2(xP, W1, W1a)
    h1p_bf, h1ai, dis, invd = _norms(degp, h1n, h1a)
    acc1 = _sc_conv64(_pack_bf16(h1p_bf), src, dst, w)
    h2p_bf, h2i = _post1(acc1, h1ai, dis, invd, b1t, W2s, W2a)
    edw = jnp.stack([dst, lax.bitcast_convert_type(w, jnp.int32)], axis=2)
    acc2 = _sc_conv128(_pack_bf16(h2p_bf), src, edw)
    z_bf, yn, ya = _post2(acc2, h2i, dis, b2t, xPt, Wd1, Wda)
    probs, degp2 = _sc_decode(_pack_bf16(z_bf), src, dst)
    yp_bf, yai, dis2 = _norms2(degp2, yn, ya)
    acc3 = _sc_conv64(_pack_bf16(yp_bf), src, dst, probs)
    out = _head(acc3, yai, dis2, bdt, Wfct, bfcr)
    return out[0, 0]
